# Initial kernel scaffold; baseline (speedup 1.0000x reference)
import functools

import jax
import jax.numpy as jnp
from jax import lax
from jax.experimental import pallas as pl
from jax.experimental.pallas import tpu as pltpu
from jax.experimental.pallas import tpu_sc as plsc

NC = 2
NS = 16
NW = NC * NS

_MESH = plsc.VectorSubcoreMesh(core_axis_name="c", subcore_axis_name="s")


@functools.lru_cache(maxsize=None)
def _make_sc_gather(Nin, M, F):
    mslice = M // NW
    CH = 128 if mslice % 128 == 0 else 88
    nch = mslice // CH

    @functools.partial(
        pl.kernel,
        out_type=jax.ShapeDtypeStruct((M, F), jnp.float32),
        mesh=_MESH,
        scratch_types=[
            pltpu.VMEM((CH,), jnp.int32),
            pltpu.VMEM((CH, F), jnp.float32),
            pltpu.SemaphoreType.DMA,
        ],
    )
    def k(table, idx, out, idx_v, rows_v, sem):
        wid = lax.axis_index("s") * NC + lax.axis_index("c")
        base = wid * mslice

        def body(j, carry):
            off = base + j * CH
            pltpu.sync_copy(idx.at[pl.ds(off, CH)], idx_v)
            pltpu.async_copy(table.at[idx_v], rows_v, sem).wait()
            pltpu.sync_copy(rows_v, out.at[pl.ds(off, CH)])
            return carry

        lax.fori_loop(0, nch, body, 0)

    return k


@functools.lru_cache(maxsize=None)
def _make_sc_degree(N, E):
    eslice = E // NW
    nech = eslice // 128
    nslice = N // NS
    ZC = nslice // 11
    nzc = nslice // ZC

    @functools.partial(
        pl.kernel,
        out_type=jax.ShapeDtypeStruct((NC, N, 8), jnp.float32),
        mesh=_MESH,
        scratch_types=[
            pltpu.VMEM_SHARED((N, 8), jnp.float32),
            pltpu.VMEM((128,), jnp.int32),
            pltpu.VMEM((128, 8), jnp.float32),
            pltpu.VMEM((ZC, 8), jnp.float32),
        ],
    )
    def k(src, ones_h, zeros_h, degp, acc, idx_v, ones_v, zbuf):
        c = lax.axis_index("c")
        s = lax.axis_index("s")
        wid = s * NC + c
        pltpu.sync_copy(ones_h, ones_v)
        pltpu.sync_copy(zeros_h.at[pl.ds(0, ZC)], zbuf)

        def zb(j, carry):
            pltpu.sync_copy(zbuf, acc.at[pl.ds(s * nslice + j * ZC, ZC)])
            return carry

        lax.fori_loop(0, nzc, zb, 0)
        plsc.subcore_barrier()

        ebase = wid * eslice

        def eb(j, carry):
            pltpu.sync_copy(src.at[pl.ds(ebase + j * 128, 128)], idx_v)
            pltpu.sync_copy(ones_v, acc.at[idx_v], add=True)
            return carry

        lax.fori_loop(0, nech, eb, 0)
        plsc.subcore_barrier()

        def wb(j, carry):
            row = s * nslice + j * ZC
            pltpu.sync_copy(acc.at[pl.ds(row, ZC)], zbuf)
            pltpu.sync_copy(zbuf, degp.at[c, pl.ds(row, ZC)])
            return carry

        lax.fori_loop(0, nzc, wb, 0)

    return k


@functools.lru_cache(maxsize=None)
def _make_sc_prop(N, E, F):
    eslice = E // NW
    nech = eslice // 128
    nslice = N // NS
    ZC = nslice // 11
    nzc = nslice // ZC

    @functools.partial(
        pl.kernel,
        out_type=jax.ShapeDtypeStruct((NC, N, F), jnp.float32),
        mesh=_MESH,
        scratch_types=[
            pltpu.VMEM_SHARED((N, F), jnp.float32),
            pltpu.VMEM((128,), jnp.int32),
            pltpu.VMEM((128,), jnp.int32),
            pltpu.VMEM((128, F), jnp.float32),
            pltpu.VMEM((ZC, F), jnp.float32),
            pltpu.SemaphoreType.DMA,
        ],
    )
    def k(hs, srcr, dstr, zeros_h, accp, acc, sidx, didx, rows_v, zbuf, sem):
        c = lax.axis_index("c")
        s = lax.axis_index("s")
        wid = s * NC + c
        pltpu.sync_copy(zeros_h.at[pl.ds(0, ZC)], zbuf)

        def zb(j, carry):
            pltpu.sync_copy(zbuf, acc.at[pl.ds(s * nslice + j * ZC, ZC)])
            return carry

        lax.fori_loop(0, nzc, zb, 0)
        plsc.subcore_barrier()

        ebase = wid * eslice

        def eb(j, carry):
            e = ebase + j * 128
            pltpu.sync_copy(srcr.at[pl.ds(e, 128)], sidx)
            pltpu.sync_copy(dstr.at[pl.ds(e, 128)], didx)
            pltpu.async_copy(hs.at[sidx], rows_v, sem).wait()
            pltpu.sync_copy(rows_v, acc.at[didx], add=True)
            return carry

        lax.fori_loop(0, nech, eb, 0)
        plsc.subcore_barrier()

        def wb(j, carry):
            row = s * nslice + j * ZC
            pltpu.sync_copy(acc.at[pl.ds(row, ZC)], zbuf)
            pltpu.sync_copy(zbuf, accp.at[c, pl.ds(row, ZC)])
            return carry

        lax.fori_loop(0, nzc, wb, 0)

    return k


def _fc_body(d_ref, w_ref, b_ref, o_ref):
    o_ref[...] = (
        jnp.dot(d_ref[...], w_ref[...], preferred_element_type=jnp.float32)
        + b_ref[...]
    )


def _tc_fc(data, fc_W, fc_b):
    B, Dout = data.shape[0], fc_W.shape[1]
    return pl.pallas_call(
        _fc_body,
        out_shape=jax.ShapeDtypeStruct((B, Dout), jnp.float32),
    )(data, fc_W, fc_b.reshape(1, Dout))


def _prep_body(degp_ref, x_ref, dinv_ref, diag_ref, hs_ref):
    deg = degp_ref[0, :, 0:1] + degp_ref[1, :, 0:1]
    pos = deg > 0.0
    dinv = jnp.where(pos, lax.rsqrt(jnp.maximum(deg, 1.0)), 0.0)
    dinv_ref[...] = dinv
    diag_ref[...] = jnp.where(pos, 0.0, -1.0)
    hs_ref[...] = dinv * x_ref[...]


def _tc_prep(degp, x, bn=1024):
    N, F = x.shape
    grid = (N // bn,)
    return pl.pallas_call(
        _prep_body,
        grid=grid,
        in_specs=[
            pl.BlockSpec((NC, bn, 8), lambda i: (0, i, 0)),
            pl.BlockSpec((bn, F), lambda i: (i, 0)),
        ],
        out_specs=[
            pl.BlockSpec((bn, 1), lambda i: (i, 0)),
            pl.BlockSpec((bn, 1), lambda i: (i, 0)),
            pl.BlockSpec((bn, F), lambda i: (i, 0)),
        ],
        out_shape=[
            jax.ShapeDtypeStruct((N, 1), jnp.float32),
            jax.ShapeDtypeStruct((N, 1), jnp.float32),
            jax.ShapeDtypeStruct((N, F), jnp.float32),
        ],
    )(degp, x)


def _comb_first_body(accp_ref, h_ref, dinv_ref, diag_ref, tx_ref, hs_ref):
    full = accp_ref[0] + accp_ref[1]
    p = diag_ref[...] * h_ref[...] - dinv_ref[...] * full
    tx_ref[...] = p
    hs_ref[...] = dinv_ref[...] * p


def _comb_rec_body(accp_ref, h_ref, tm2_ref, dinv_ref, diag_ref, tx_ref, hs_ref):
    full = accp_ref[0] + accp_ref[1]
    p = 2.0 * (diag_ref[...] * h_ref[...] - dinv_ref[...] * full) - tm2_ref[...]
    tx_ref[...] = p
    hs_ref[...] = dinv_ref[...] * p


def _tc_combine(accp, h, tm2, dinv, diag, bn=1024):
    N, F = h.shape
    grid = (N // bn,)
    nf = pl.BlockSpec((bn, F), lambda i: (i, 0))
    n1 = pl.BlockSpec((bn, 1), lambda i: (i, 0))
    accs = pl.BlockSpec((NC, bn, F), lambda i: (0, i, 0))
    if tm2 is None:
        body, ins, specs = _comb_first_body, (accp, h, dinv, diag), [accs, nf, n1, n1]
    else:
        body, ins, specs = (_comb_rec_body, (accp, h, tm2, dinv, diag),
                            [accs, nf, nf, n1, n1])
    return pl.pallas_call(
        body,
        grid=grid,
        in_specs=specs,
        out_specs=[nf, nf],
        out_shape=[
            jax.ShapeDtypeStruct((N, F), jnp.float32),
            jax.ShapeDtypeStruct((N, F), jnp.float32),
        ],
    )(*ins)


def _make_mm_body(K, relu):
    def body(*refs):
        t_refs, w_ref, b_ref, o_ref = refs[:K], refs[K], refs[K + 1], refs[K + 2]
        acc = b_ref[...]
        for k in range(K):
            acc = acc + jnp.dot(
                t_refs[k][...], w_ref[k], preferred_element_type=jnp.float32
            )
        o_ref[...] = jnp.maximum(acc, 0.0) if relu else acc

    return body


def _tc_mm(txs, W, b, relu, bn=1024):
    N, Fin = txs[0].shape
    K, _, Fout = W.shape
    grid = (N // bn,)
    nf = pl.BlockSpec((bn, Fin), lambda i: (i, 0))
    return pl.pallas_call(
        _make_mm_body(K, relu),
        grid=grid,
        in_specs=[nf] * K
        + [
            pl.BlockSpec((K, Fin, Fout), lambda i: (0, 0, 0)),
            pl.BlockSpec((1, Fout), lambda i: (0, 0)),
        ],
        out_specs=pl.BlockSpec((bn, Fout), lambda i: (i, 0)),
        out_shape=jax.ShapeDtypeStruct((N, Fout), jnp.float32),
    )(*txs, W, b.reshape(1, Fout))


def _level(x, up_idx, edge_index, W, b, relu):
    Nin, Fin = x.shape
    N = up_idx.shape[0]
    E = edge_index.shape[1]
    K = W.shape[0]
    src = edge_index[0]
    dst = edge_index[1]

    x = _make_sc_gather(Nin, N, Fin)(x, up_idx)

    ones8 = jnp.ones((128, 8), jnp.float32)
    zeros8 = jnp.zeros((N, 8), jnp.float32)
    zerosF = jnp.zeros((N, Fin), jnp.float32)

    degp = _make_sc_degree(N, E)(src, ones8, zeros8)
    dinv, diag, hs = _tc_prep(degp, x)

    prop = _make_sc_prop(N, E, Fin)
    txs = [x]
    accp = prop(hs, src, dst, zerosF)
    tx, hs = _tc_combine(accp, x, None, dinv, diag)
    txs.append(tx)
    for _ in range(2, K):
        accp = prop(hs, src, dst, zerosF)
        tx, hs = _tc_combine(accp, txs[-1], txs[-2], dinv, diag)
        txs.append(tx)

    return _tc_mm(txs, W, b, relu)


def kernel(data, up_idx_4to3, edge_index_3, up_idx_3to2, edge_index_2,
           up_idx_2to1, edge_index_1, up_idx_1to0, edge_index_0,
           fc_W, fc_b, W1, b1, W2, b2, W3, b3, W4, b4):
    x = _tc_fc(data, fc_W, fc_b)
    x = x.reshape(-1, 32)
    x = _level(x, up_idx_4to3, edge_index_3, W1, b1, True)
    x = _level(x, up_idx_3to2, edge_index_2, W2, b2, True)
    x = _level(x, up_idx_2to1, edge_index_1, W3, b3, True)
    x = _level(x, up_idx_1to0, edge_index_0, W4, b4, False)
    return x


# speedup vs baseline: 14.5716x; 14.5716x over previous
import functools

import jax
import jax.numpy as jnp
from jax import lax
from jax.experimental import pallas as pl
from jax.experimental.pallas import tpu as pltpu
from jax.experimental.pallas import tpu_sc as plsc

NC = 2
NS = 16
NW = NC * NS

_MESH = plsc.VectorSubcoreMesh(core_axis_name="c", subcore_axis_name="s")
_SC_PARAMS = pltpu.CompilerParams(use_tc_tiling_on_sc=False)


@functools.lru_cache(maxsize=None)
def _make_sc_gather(Nin, M, F):
    mslice = M // NW
    CH = 128 if mslice % 128 == 0 else 88
    nch = mslice // CH

    @functools.partial(
        pl.kernel,
        out_type=jax.ShapeDtypeStruct((M, F), jnp.float32),
        mesh=_MESH,
        compiler_params=_SC_PARAMS,
        scratch_types=[
            pltpu.VMEM((CH,), jnp.int32),
            pltpu.VMEM((CH, F), jnp.float32),
            pltpu.SemaphoreType.DMA,
        ],
    )
    def k(table, idx, out, idx_v, rows_v, sem):
        wid = lax.axis_index("s") * NC + lax.axis_index("c")
        base = wid * mslice

        def body(j, carry):
            off = base + j * CH
            pltpu.sync_copy(idx.at[pl.ds(off, CH)], idx_v)
            pltpu.async_copy(table.at[idx_v], rows_v, sem).wait()
            pltpu.sync_copy(rows_v, out.at[pl.ds(off, CH)])
            return carry

        lax.fori_loop(0, nch, body, 0)

    return k


@functools.lru_cache(maxsize=None)
def _make_sc_degree(N, E):
    eslice = E // NW
    nech = eslice // 128
    nslice = N // NS
    ZC = nslice // 11
    nzc = nslice // ZC

    @functools.partial(
        pl.kernel,
        out_type=jax.ShapeDtypeStruct((NC, N, 8), jnp.float32),
        mesh=_MESH,
        compiler_params=_SC_PARAMS,
        scratch_types=[
            pltpu.VMEM_SHARED((N, 8), jnp.float32),
            pltpu.VMEM((128,), jnp.int32),
            pltpu.VMEM((128, 8), jnp.float32),
            pltpu.VMEM((ZC, 8), jnp.float32),
        ],
    )
    def k(src, ones_h, zeros_h, degp, acc, idx_v, ones_v, zbuf):
        c = lax.axis_index("c")
        s = lax.axis_index("s")
        wid = s * NC + c
        pltpu.sync_copy(ones_h, ones_v)
        pltpu.sync_copy(zeros_h.at[pl.ds(0, ZC)], zbuf)

        def zb(j, carry):
            pltpu.sync_copy(zbuf, acc.at[pl.ds(s * nslice + j * ZC, ZC)])
            return carry

        lax.fori_loop(0, nzc, zb, 0)
        plsc.subcore_barrier()

        ebase = wid * eslice

        def eb(j, carry):
            pltpu.sync_copy(src.at[pl.ds(ebase + j * 128, 128)], idx_v)
            pltpu.sync_copy(ones_v, acc.at[idx_v], add=True)
            return carry

        lax.fori_loop(0, nech, eb, 0)
        plsc.subcore_barrier()

        def wb(j, carry):
            row = s * nslice + j * ZC
            pltpu.sync_copy(acc.at[pl.ds(row, ZC)], zbuf)
            pltpu.sync_copy(zbuf, degp.at[c, pl.ds(row, ZC)])
            return carry

        lax.fori_loop(0, nzc, wb, 0)

    return k


@functools.lru_cache(maxsize=None)
def _make_sc_prop(N, E, F):
    eslice = E // NW
    nech = eslice // 128
    nslice = N // NS
    ZC = nslice // 11
    nzc = nslice // ZC

    @functools.partial(
        pl.kernel,
        out_type=jax.ShapeDtypeStruct((NC, N, F), jnp.float32),
        mesh=_MESH,
        compiler_params=_SC_PARAMS,
        scratch_types=[
            pltpu.VMEM_SHARED((N, F), jnp.float32),
            pltpu.VMEM((128,), jnp.int32),
            pltpu.VMEM((128,), jnp.int32),
            pltpu.VMEM((128, F), jnp.float32),
            pltpu.VMEM((ZC, F), jnp.float32),
            pltpu.SemaphoreType.DMA,
        ],
    )
    def k(hs, srcr, dstr, zeros_h, accp, acc, sidx, didx, rows_v, zbuf, sem):
        c = lax.axis_index("c")
        s = lax.axis_index("s")
        wid = s * NC + c
        pltpu.sync_copy(zeros_h.at[pl.ds(0, ZC)], zbuf)

        def zb(j, carry):
            pltpu.sync_copy(zbuf, acc.at[pl.ds(s * nslice + j * ZC, ZC)])
            return carry

        lax.fori_loop(0, nzc, zb, 0)
        plsc.subcore_barrier()

        ebase = wid * eslice

        def eb(j, carry):
            e = ebase + j * 128
            pltpu.sync_copy(srcr.at[pl.ds(e, 128)], sidx)
            pltpu.sync_copy(dstr.at[pl.ds(e, 128)], didx)
            pltpu.async_copy(hs.at[sidx], rows_v, sem).wait()
            pltpu.sync_copy(rows_v, acc.at[didx], add=True)
            return carry

        lax.fori_loop(0, nech, eb, 0)
        plsc.subcore_barrier()

        def wb(j, carry):
            row = s * nslice + j * ZC
            pltpu.sync_copy(acc.at[pl.ds(row, ZC)], zbuf)
            pltpu.sync_copy(zbuf, accp.at[c, pl.ds(row, ZC)])
            return carry

        lax.fori_loop(0, nzc, wb, 0)

    return k


def _fc_body(d_ref, w_ref, b_ref, o_ref):
    o_ref[...] = (
        jnp.dot(d_ref[...], w_ref[...], preferred_element_type=jnp.float32)
        + b_ref[...]
    )


def _tc_fc(data, fc_W, fc_b):
    B, Dout = data.shape[0], fc_W.shape[1]
    return pl.pallas_call(
        _fc_body,
        out_shape=jax.ShapeDtypeStruct((B, Dout), jnp.float32),
    )(data, fc_W, fc_b.reshape(1, Dout))


def _prep_body(degp_ref, x_ref, dinv_ref, diag_ref, hs_ref):
    deg = degp_ref[0, :, 0:1] + degp_ref[1, :, 0:1]
    pos = deg > 0.0
    dinv = jnp.where(pos, lax.rsqrt(jnp.maximum(deg, 1.0)), 0.0)
    dinv_ref[...] = dinv
    diag_ref[...] = jnp.where(pos, 0.0, -1.0)
    hs_ref[...] = dinv * x_ref[...]


def _tc_prep(degp, x, bn=1024):
    N, F = x.shape
    grid = (N // bn,)
    return pl.pallas_call(
        _prep_body,
        grid=grid,
        in_specs=[
            pl.BlockSpec((NC, bn, 8), lambda i: (0, i, 0)),
            pl.BlockSpec((bn, F), lambda i: (i, 0)),
        ],
        out_specs=[
            pl.BlockSpec((bn, 1), lambda i: (i, 0)),
            pl.BlockSpec((bn, 1), lambda i: (i, 0)),
            pl.BlockSpec((bn, F), lambda i: (i, 0)),
        ],
        out_shape=[
            jax.ShapeDtypeStruct((N, 1), jnp.float32),
            jax.ShapeDtypeStruct((N, 1), jnp.float32),
            jax.ShapeDtypeStruct((N, F), jnp.float32),
        ],
    )(degp, x)


def _comb_first_body(accp_ref, h_ref, dinv_ref, diag_ref, tx_ref, hs_ref):
    full = accp_ref[0] + accp_ref[1]
    p = diag_ref[...] * h_ref[...] - dinv_ref[...] * full
    tx_ref[...] = p
    hs_ref[...] = dinv_ref[...] * p


def _comb_rec_body(accp_ref, h_ref, tm2_ref, dinv_ref, diag_ref, tx_ref, hs_ref):
    full = accp_ref[0] + accp_ref[1]
    p = 2.0 * (diag_ref[...] * h_ref[...] - dinv_ref[...] * full) - tm2_ref[...]
    tx_ref[...] = p
    hs_ref[...] = dinv_ref[...] * p


def _tc_combine(accp, h, tm2, dinv, diag, bn=1024):
    N, F = h.shape
    grid = (N // bn,)
    nf = pl.BlockSpec((bn, F), lambda i: (i, 0))
    n1 = pl.BlockSpec((bn, 1), lambda i: (i, 0))
    accs = pl.BlockSpec((NC, bn, F), lambda i: (0, i, 0))
    if tm2 is None:
        body, ins, specs = _comb_first_body, (accp, h, dinv, diag), [accs, nf, n1, n1]
    else:
        body, ins, specs = (_comb_rec_body, (accp, h, tm2, dinv, diag),
                            [accs, nf, nf, n1, n1])
    return pl.pallas_call(
        body,
        grid=grid,
        in_specs=specs,
        out_specs=[nf, nf],
        out_shape=[
            jax.ShapeDtypeStruct((N, F), jnp.float32),
            jax.ShapeDtypeStruct((N, F), jnp.float32),
        ],
    )(*ins)


def _make_mm_body(K, relu):
    def body(*refs):
        t_refs, w_ref, b_ref, o_ref = refs[:K], refs[K], refs[K + 1], refs[K + 2]
        acc = b_ref[...]
        for k in range(K):
            acc = acc + jnp.dot(
                t_refs[k][...], w_ref[k], preferred_element_type=jnp.float32
            )
        o_ref[...] = jnp.maximum(acc, 0.0) if relu else acc

    return body


def _tc_mm(txs, W, b, relu, bn=1024):
    N, Fin = txs[0].shape
    K, _, Fout = W.shape
    grid = (N // bn,)
    nf = pl.BlockSpec((bn, Fin), lambda i: (i, 0))
    return pl.pallas_call(
        _make_mm_body(K, relu),
        grid=grid,
        in_specs=[nf] * K
        + [
            pl.BlockSpec((K, Fin, Fout), lambda i: (0, 0, 0)),
            pl.BlockSpec((1, Fout), lambda i: (0, 0)),
        ],
        out_specs=pl.BlockSpec((bn, Fout), lambda i: (i, 0)),
        out_shape=jax.ShapeDtypeStruct((N, Fout), jnp.float32),
    )(*txs, W, b.reshape(1, Fout))


def _level(x, up_idx, edge_index, W, b, relu):
    Nin, Fin = x.shape
    N = up_idx.shape[0]
    E = edge_index.shape[1]
    K = W.shape[0]
    src = edge_index[0]
    dst = edge_index[1]

    x = _make_sc_gather(Nin, N, Fin)(x, up_idx)

    ones8 = jnp.ones((128, 8), jnp.float32)
    zeros8 = jnp.zeros((N, 8), jnp.float32)
    zerosF = jnp.zeros((N, Fin), jnp.float32)

    degp = _make_sc_degree(N, E)(src, ones8, zeros8)
    dinv, diag, hs = _tc_prep(degp, x)

    prop = _make_sc_prop(N, E, Fin)
    txs = [x]
    accp = prop(hs, src, dst, zerosF)
    tx, hs = _tc_combine(accp, x, None, dinv, diag)
    txs.append(tx)
    for _ in range(2, K):
        accp = prop(hs, src, dst, zerosF)
        tx, hs = _tc_combine(accp, txs[-1], txs[-2], dinv, diag)
        txs.append(tx)

    return _tc_mm(txs, W, b, relu)


def kernel(data, up_idx_4to3, edge_index_3, up_idx_3to2, edge_index_2,
           up_idx_2to1, edge_index_1, up_idx_1to0, edge_index_0,
           fc_W, fc_b, W1, b1, W2, b2, W3, b3, W4, b4):
    x = _tc_fc(data, fc_W, fc_b)
    x = x.reshape(-1, 32)
    x = _level(x, up_idx_4to3, edge_index_3, W1, b1, True)
    x = _level(x, up_idx_3to2, edge_index_2, W2, b2, True)
    x = _level(x, up_idx_2to1, edge_index_1, W3, b3, True)
    x = _level(x, up_idx_1to0, edge_index_0, W4, b4, False)
    return x


# speedup vs baseline: 30.4077x; 2.0868x over previous
import functools

import jax
import jax.numpy as jnp
from jax import lax
from jax.experimental import pallas as pl
from jax.experimental.pallas import tpu as pltpu
from jax.experimental.pallas import tpu_sc as plsc

NC = 2
NS = 16
NW = NC * NS

_MESH = plsc.VectorSubcoreMesh(core_axis_name="c", subcore_axis_name="s")
_SC_PARAMS = pltpu.CompilerParams(use_tc_tiling_on_sc=False)


@functools.lru_cache(maxsize=None)
def _make_sc_gather(Nin, M, F):
    mslice = M // NW
    CH = 88
    nbg = mslice // CH

    @functools.partial(
        pl.kernel,
        out_type=jax.ShapeDtypeStruct((M, F), jnp.float32),
        mesh=_MESH,
        compiler_params=_SC_PARAMS,
        scratch_types=[
            pltpu.VMEM((mslice,), jnp.int32),
            pltpu.VMEM((mslice, F), jnp.float32),
            pltpu.SemaphoreType.DMA,
        ],
    )
    def k(table, idx, out, idx_v, rows_v, sem):
        wid = lax.axis_index("s") * NC + lax.axis_index("c")
        base = wid * mslice
        pltpu.sync_copy(idx.at[pl.ds(base, mslice)], idx_v)
        descs = [
            pltpu.async_copy(
                table.at[idx_v.at[pl.ds(b * CH, CH)]],
                rows_v.at[pl.ds(b * CH, CH)],
                sem,
            )
            for b in range(nbg)
        ]
        for d in descs:
            d.wait()
        pltpu.sync_copy(rows_v, out.at[pl.ds(base, mslice)])

    return k


_NBR = 11


@functools.lru_cache(maxsize=None)
def _make_sc_degree(N, E):
    erows = E // 128 // NW
    nwav = erows // _NBR
    nslice = N // NS
    ZC = nslice // 11
    nzc = nslice // ZC

    @functools.partial(
        pl.kernel,
        out_type=jax.ShapeDtypeStruct((NC, N, 8), jnp.float32),
        mesh=_MESH,
        compiler_params=_SC_PARAMS,
        scratch_types=[
            pltpu.VMEM_SHARED((N, 8), jnp.float32),
            pltpu.VMEM((_NBR, 128), jnp.int32),
            pltpu.VMEM((128, 8), jnp.float32),
            pltpu.VMEM((ZC, 8), jnp.float32),
            pltpu.SemaphoreType.DMA,
        ],
    )
    def k(src, ones_h, zeros_h, degp, acc, dblk, ones_v, zbuf, sem_s):
        c = lax.axis_index("c")
        s = lax.axis_index("s")
        wid = s * NC + c
        pltpu.sync_copy(ones_h, ones_v)
        pltpu.sync_copy(zeros_h.at[pl.ds(0, ZC)], zbuf)

        def zb(j, carry):
            pltpu.sync_copy(zbuf, acc.at[pl.ds(s * nslice + j * ZC, ZC)])
            return carry

        lax.fori_loop(0, nzc, zb, 0)
        plsc.subcore_barrier()

        rbase = wid * erows

        def eb(w, carry):
            pltpu.sync_copy(src.at[pl.ds(rbase + w * _NBR, _NBR)], dblk)
            descs = [
                pltpu.async_copy(ones_v, acc.at[dblk.at[b]], sem_s, add=True)
                for b in range(_NBR)
            ]
            for d in descs:
                d.wait()
            return carry

        lax.fori_loop(0, nwav, eb, 0)
        plsc.subcore_barrier()

        def wb(j, carry):
            row = s * nslice + j * ZC
            pltpu.sync_copy(acc.at[pl.ds(row, ZC)], zbuf)
            pltpu.sync_copy(zbuf, degp.at[c, pl.ds(row, ZC)])
            return carry

        lax.fori_loop(0, nzc, wb, 0)

    return k


@functools.lru_cache(maxsize=None)
def _make_sc_prop(N, E, F):
    erows = E // 128 // NW
    nwav = erows // _NBR
    nslice = N // NS
    ZC = nslice // 11
    nzc = nslice // ZC

    @functools.partial(
        pl.kernel,
        out_type=jax.ShapeDtypeStruct((NC, N, F), jnp.float32),
        mesh=_MESH,
        compiler_params=_SC_PARAMS,
        scratch_types=[
            pltpu.VMEM_SHARED((N, F), jnp.float32),
            pltpu.VMEM((_NBR, 128), jnp.int32),
            pltpu.VMEM((_NBR, 128), jnp.int32),
            pltpu.VMEM((_NBR * 128, F), jnp.float32),
            pltpu.VMEM((ZC, F), jnp.float32),
            pltpu.SemaphoreType.DMA,
            pltpu.SemaphoreType.DMA,
        ],
    )
    def k(hs, srcr, dstr, zeros_h, accp, acc, sblk, dblk, rows_v, zbuf,
          sem_g, sem_s):
        c = lax.axis_index("c")
        s = lax.axis_index("s")
        wid = s * NC + c
        pltpu.sync_copy(zeros_h.at[pl.ds(0, ZC)], zbuf)

        def zb(j, carry):
            pltpu.sync_copy(zbuf, acc.at[pl.ds(s * nslice + j * ZC, ZC)])
            return carry

        lax.fori_loop(0, nzc, zb, 0)
        plsc.subcore_barrier()

        rbase = wid * erows

        def eb(w, carry):
            r = rbase + w * _NBR
            pltpu.sync_copy(srcr.at[pl.ds(r, _NBR)], sblk)
            pltpu.sync_copy(dstr.at[pl.ds(r, _NBR)], dblk)
            gds = [
                pltpu.async_copy(
                    hs.at[sblk.at[b]],
                    rows_v.at[pl.ds(b * 128, 128)],
                    sem_g,
                )
                for b in range(_NBR)
            ]
            for d in gds:
                d.wait()
            sds = [
                pltpu.async_copy(
                    rows_v.at[pl.ds(b * 128, 128)],
                    acc.at[dblk.at[b]],
                    sem_s,
                    add=True,
                )
                for b in range(_NBR)
            ]
            for d in sds:
                d.wait()
            return carry

        lax.fori_loop(0, nwav, eb, 0)
        plsc.subcore_barrier()

        def wb(j, carry):
            row = s * nslice + j * ZC
            pltpu.sync_copy(acc.at[pl.ds(row, ZC)], zbuf)
            pltpu.sync_copy(zbuf, accp.at[c, pl.ds(row, ZC)])
            return carry

        lax.fori_loop(0, nzc, wb, 0)

    return k


def _fc_body(d_ref, w_ref, b_ref, o_ref):
    o_ref[...] = (
        jnp.dot(d_ref[...], w_ref[...], preferred_element_type=jnp.float32)
        + b_ref[...]
    )


def _tc_fc(data, fc_W, fc_b):
    B, Dout = data.shape[0], fc_W.shape[1]
    return pl.pallas_call(
        _fc_body,
        out_shape=jax.ShapeDtypeStruct((B, Dout), jnp.float32),
    )(data, fc_W, fc_b.reshape(1, Dout))


def _prep_body(degp_ref, x_ref, dinv_ref, diag_ref, hs_ref):
    deg = degp_ref[0, :, 0:1] + degp_ref[1, :, 0:1]
    pos = deg > 0.0
    dinv = jnp.where(pos, lax.rsqrt(jnp.maximum(deg, 1.0)), 0.0)
    dinv_ref[...] = dinv
    diag_ref[...] = jnp.where(pos, 0.0, -1.0)
    hs_ref[...] = dinv * x_ref[...]


def _tc_prep(degp, x, bn=1024):
    N, F = x.shape
    grid = (N // bn,)
    return pl.pallas_call(
        _prep_body,
        grid=grid,
        in_specs=[
            pl.BlockSpec((NC, bn, 8), lambda i: (0, i, 0)),
            pl.BlockSpec((bn, F), lambda i: (i, 0)),
        ],
        out_specs=[
            pl.BlockSpec((bn, 1), lambda i: (i, 0)),
            pl.BlockSpec((bn, 1), lambda i: (i, 0)),
            pl.BlockSpec((bn, F), lambda i: (i, 0)),
        ],
        out_shape=[
            jax.ShapeDtypeStruct((N, 1), jnp.float32),
            jax.ShapeDtypeStruct((N, 1), jnp.float32),
            jax.ShapeDtypeStruct((N, F), jnp.float32),
        ],
    )(degp, x)


def _comb_first_body(accp_ref, h_ref, dinv_ref, diag_ref, tx_ref, hs_ref):
    full = accp_ref[0] + accp_ref[1]
    p = diag_ref[...] * h_ref[...] - dinv_ref[...] * full
    tx_ref[...] = p
    hs_ref[...] = dinv_ref[...] * p


def _comb_rec_body(accp_ref, h_ref, tm2_ref, dinv_ref, diag_ref, tx_ref, hs_ref):
    full = accp_ref[0] + accp_ref[1]
    p = 2.0 * (diag_ref[...] * h_ref[...] - dinv_ref[...] * full) - tm2_ref[...]
    tx_ref[...] = p
    hs_ref[...] = dinv_ref[...] * p


def _tc_combine(accp, h, tm2, dinv, diag, bn=1024):
    N, F = h.shape
    grid = (N // bn,)
    nf = pl.BlockSpec((bn, F), lambda i: (i, 0))
    n1 = pl.BlockSpec((bn, 1), lambda i: (i, 0))
    accs = pl.BlockSpec((NC, bn, F), lambda i: (0, i, 0))
    if tm2 is None:
        body, ins, specs = _comb_first_body, (accp, h, dinv, diag), [accs, nf, n1, n1]
    else:
        body, ins, specs = (_comb_rec_body, (accp, h, tm2, dinv, diag),
                            [accs, nf, nf, n1, n1])
    return pl.pallas_call(
        body,
        grid=grid,
        in_specs=specs,
        out_specs=[nf, nf],
        out_shape=[
            jax.ShapeDtypeStruct((N, F), jnp.float32),
            jax.ShapeDtypeStruct((N, F), jnp.float32),
        ],
    )(*ins)


def _make_mm_body(K, relu):
    def body(*refs):
        t_refs, w_ref, b_ref, o_ref = refs[:K], refs[K], refs[K + 1], refs[K + 2]
        acc = b_ref[...]
        for k in range(K):
            acc = acc + jnp.dot(
                t_refs[k][...], w_ref[k], preferred_element_type=jnp.float32
            )
        o_ref[...] = jnp.maximum(acc, 0.0) if relu else acc

    return body


def _tc_mm(txs, W, b, relu, bn=1024):
    N, Fin = txs[0].shape
    K, _, Fout = W.shape
    grid = (N // bn,)
    nf = pl.BlockSpec((bn, Fin), lambda i: (i, 0))
    return pl.pallas_call(
        _make_mm_body(K, relu),
        grid=grid,
        in_specs=[nf] * K
        + [
            pl.BlockSpec((K, Fin, Fout), lambda i: (0, 0, 0)),
            pl.BlockSpec((1, Fout), lambda i: (0, 0)),
        ],
        out_specs=pl.BlockSpec((bn, Fout), lambda i: (i, 0)),
        out_shape=jax.ShapeDtypeStruct((N, Fout), jnp.float32),
    )(*txs, W, b.reshape(1, Fout))


def _level(x, up_idx, edge_index, W, b, relu):
    Nin, Fin = x.shape
    N = up_idx.shape[0]
    E = edge_index.shape[1]
    K = W.shape[0]
    src = edge_index[0].reshape(E // 128, 128)
    dst = edge_index[1].reshape(E // 128, 128)

    x = _make_sc_gather(Nin, N, Fin)(x, up_idx)

    ones8 = jnp.ones((128, 8), jnp.float32)
    zeros8 = jnp.zeros((N, 8), jnp.float32)
    zerosF = jnp.zeros((N, Fin), jnp.float32)

    degp = _make_sc_degree(N, E)(src, ones8, zeros8)
    dinv, diag, hs = _tc_prep(degp, x)

    prop = _make_sc_prop(N, E, Fin)
    txs = [x]
    accp = prop(hs, src, dst, zerosF)
    tx, hs = _tc_combine(accp, x, None, dinv, diag)
    txs.append(tx)
    for _ in range(2, K):
        accp = prop(hs, src, dst, zerosF)
        tx, hs = _tc_combine(accp, txs[-1], txs[-2], dinv, diag)
        txs.append(tx)

    return _tc_mm(txs, W, b, relu)


def kernel(data, up_idx_4to3, edge_index_3, up_idx_3to2, edge_index_2,
           up_idx_2to1, edge_index_1, up_idx_1to0, edge_index_0,
           fc_W, fc_b, W1, b1, W2, b2, W3, b3, W4, b4):
    x = _tc_fc(data, fc_W, fc_b)
    x = x.reshape(-1, 32)
    x = _level(x, up_idx_4to3, edge_index_3, W1, b1, True)
    x = _level(x, up_idx_3to2, edge_index_2, W2, b2, True)
    x = _level(x, up_idx_2to1, edge_index_1, W3, b3, True)
    x = _level(x, up_idx_1to0, edge_index_0, W4, b4, False)
    return x


# speedup vs baseline: 31.2207x; 1.0267x over previous
import functools

import jax
import jax.numpy as jnp
from jax import lax
from jax.experimental import pallas as pl
from jax.experimental.pallas import tpu as pltpu
from jax.experimental.pallas import tpu_sc as plsc

NC = 2
NS = 16
NW = NC * NS

_MESH = plsc.VectorSubcoreMesh(core_axis_name="c", subcore_axis_name="s")
_SC_PARAMS = pltpu.CompilerParams(use_tc_tiling_on_sc=False)


@functools.lru_cache(maxsize=None)
def _make_sc_gather(Nin, M, F):
    mslice = M // NW
    CH = 88
    nbg = mslice // CH

    @functools.partial(
        pl.kernel,
        out_type=jax.ShapeDtypeStruct((M, F), jnp.float32),
        mesh=_MESH,
        compiler_params=_SC_PARAMS,
        scratch_types=[
            pltpu.VMEM((mslice,), jnp.int32),
            pltpu.VMEM((mslice, F), jnp.float32),
            pltpu.SemaphoreType.DMA,
        ],
    )
    def k(table, idx, out, idx_v, rows_v, sem):
        wid = lax.axis_index("s") * NC + lax.axis_index("c")
        base = wid * mslice
        pltpu.sync_copy(idx.at[pl.ds(base, mslice)], idx_v)
        descs = [
            pltpu.async_copy(
                table.at[idx_v.at[pl.ds(b * CH, CH)]],
                rows_v.at[pl.ds(b * CH, CH)],
                sem,
            )
            for b in range(nbg)
        ]
        for d in descs:
            d.wait()
        pltpu.sync_copy(rows_v, out.at[pl.ds(base, mslice)])

    return k


_NBR = 11


@functools.lru_cache(maxsize=None)
def _make_sc_deg_all(levels, xshape):
    Nmax = max(n for n, _ in levels)

    @functools.partial(
        pl.kernel,
        out_type=[jax.ShapeDtypeStruct((NC, n, 8), jnp.float32)
                  for n, _ in levels],
        mesh=_MESH,
        compiler_params=_SC_PARAMS,
        scratch_types=[
            pltpu.VMEM_SHARED((Nmax, 8), jnp.float32),
            pltpu.VMEM((_NBR, 128), jnp.int32),
            pltpu.VMEM((128, 8), jnp.float32),
            pltpu.VMEM((512, 8), jnp.float32),
            pltpu.VMEM((512, 8), jnp.float32),
            pltpu.SemaphoreType.DMA,
        ],
    )
    def k(*refs):
        nl = len(levels)
        srcs = refs[:nl]
        ones_h, zeros_h = refs[nl], refs[nl + 1]
        outs = refs[nl + 3:nl + 3 + nl]
        acc, dblk, ones_v, zbuf, obuf, sem_s = refs[nl + 3 + nl:]
        c = lax.axis_index("c")
        s = lax.axis_index("s")
        wid = s * NC + c
        pltpu.sync_copy(ones_h, ones_v)
        pltpu.sync_copy(zeros_h, zbuf)

        for (N, E), src, degp in zip(levels, srcs, outs):
            erows = E // 128 // NW
            nwav = erows // _NBR
            nslice = N // NS
            ZC = nslice // 11

            def zb(j, carry, s=s, nslice=nslice, ZC=ZC, acc=acc):
                pltpu.sync_copy(zbuf.at[pl.ds(0, ZC)],
                                acc.at[pl.ds(s * nslice + j * ZC, ZC)])
                return carry

            lax.fori_loop(0, 11, zb, 0)
            plsc.subcore_barrier()

            rbase = wid * erows

            def eb(w, carry, src=src, rbase=rbase):
                pltpu.sync_copy(src.at[pl.ds(rbase + w * _NBR, _NBR)], dblk)
                descs = [
                    pltpu.async_copy(ones_v, acc.at[dblk.at[b]], sem_s,
                                     add=True)
                    for b in range(_NBR)
                ]
                for d in descs:
                    d.wait()
                return carry

            lax.fori_loop(0, nwav, eb, 0)
            plsc.subcore_barrier()

            def wb(j, carry, s=s, nslice=nslice, ZC=ZC, degp=degp, c=c):
                row = s * nslice + j * ZC
                pltpu.sync_copy(acc.at[pl.ds(row, ZC)],
                                obuf.at[pl.ds(0, ZC)])
                pltpu.sync_copy(obuf.at[pl.ds(0, ZC)],
                                degp.at[c, pl.ds(row, ZC)])
                return carry

            lax.fori_loop(0, 11, wb, 0)
            plsc.subcore_barrier()

    return k


@functools.lru_cache(maxsize=None)
def _make_sc_prop(N, E, F):
    erows = E // 128 // NW
    nslice = N // NS
    ZC = nslice // 11
    nzc = nslice // ZC

    NBR = None
    for nbr in (11, 8, 4, 2, 1):
        if erows % nbr or (erows // nbr) % 2:
            continue
        tile_words = 2 * nbr * 128 * F + 4 * nbr * 128 + ZC * F + 4096
        if N * F + NS * tile_words <= 1_950_000:
            NBR = nbr
            break
    nwav = erows // NBR
    T = nwav // 2

    @functools.partial(
        pl.kernel,
        out_type=jax.ShapeDtypeStruct((NC, N, F), jnp.float32),
        mesh=_MESH,
        compiler_params=_SC_PARAMS,
        scratch_types=[
            pltpu.VMEM_SHARED((N, F), jnp.float32),
            pltpu.VMEM((NBR, 128), jnp.int32),
            pltpu.VMEM((NBR, 128), jnp.int32),
            pltpu.VMEM((NBR, 128), jnp.int32),
            pltpu.VMEM((NBR, 128), jnp.int32),
            pltpu.VMEM((NBR * 128, F), jnp.float32),
            pltpu.VMEM((NBR * 128, F), jnp.float32),
            pltpu.VMEM((ZC, F), jnp.float32),
            pltpu.SemaphoreType.DMA,
            pltpu.SemaphoreType.DMA,
            pltpu.SemaphoreType.DMA,
            pltpu.SemaphoreType.DMA,
        ],
    )
    def k(hs, srcr, dstr, zeros_h, accp, acc, sblk0, dblk0, sblk1, dblk1,
          rows0, rows1, zbuf, sem_g0, sem_g1, sem_s0, sem_s1):
        c = lax.axis_index("c")
        s = lax.axis_index("s")
        wid = s * NC + c
        pltpu.sync_copy(zeros_h.at[pl.ds(0, ZC)], zbuf)

        def zb(j, carry):
            pltpu.sync_copy(zbuf, acc.at[pl.ds(s * nslice + j * ZC, ZC)])
            return carry

        lax.fori_loop(0, nzc, zb, 0)
        plsc.subcore_barrier()

        rbase = wid * erows
        bufs = ((sblk0, dblk0, rows0, sem_g0, sem_s0),
                (sblk1, dblk1, rows1, sem_g1, sem_s1))

        def load_and_fire(wrow, sblk, dblk, rows, sem_g):
            pltpu.sync_copy(srcr.at[pl.ds(wrow, NBR)], sblk)
            pltpu.sync_copy(dstr.at[pl.ds(wrow, NBR)], dblk)
            for b in range(NBR):
                pltpu.async_copy(hs.at[sblk.at[b]],
                                 rows.at[pl.ds(b * 128, 128)], sem_g)

        load_and_fire(rbase, sblk0, dblk0, rows0, sem_g0)
        load_and_fire(rbase + NBR, sblk1, dblk1, rows1, sem_g1)

        def eb(t, carry):
            for p, (sblk, dblk, rows, sem_g, sem_s) in enumerate(bufs):
                for b in range(NBR):
                    pltpu.make_async_copy(
                        hs.at[sblk.at[b]],
                        rows.at[pl.ds(b * 128, 128)], sem_g).wait()
                sds = [
                    pltpu.async_copy(rows.at[pl.ds(b * 128, 128)],
                                     acc.at[dblk.at[b]], sem_s, add=True)
                    for b in range(NBR)
                ]
                for d in sds:
                    d.wait()

                @pl.when(t < T - 1)
                def _():
                    load_and_fire(rbase + (2 * t + p + 2) * NBR,
                                  sblk, dblk, rows, sem_g)

            return carry

        lax.fori_loop(0, T, eb, 0)
        plsc.subcore_barrier()

        def wb(j, carry):
            row = s * nslice + j * ZC
            pltpu.sync_copy(acc.at[pl.ds(row, ZC)], zbuf)
            pltpu.sync_copy(zbuf, accp.at[c, pl.ds(row, ZC)])
            return carry

        lax.fori_loop(0, nzc, wb, 0)

    return k


def _fc_body(d_ref, w_ref, b_ref, o_ref):
    o_ref[...] = (
        jnp.dot(d_ref[...], w_ref[...], preferred_element_type=jnp.float32)
        + b_ref[...]
    )


def _tc_fc(data, fc_W, fc_b):
    B, Dout = data.shape[0], fc_W.shape[1]
    return pl.pallas_call(
        _fc_body,
        out_shape=jax.ShapeDtypeStruct((B, Dout), jnp.float32),
    )(data, fc_W, fc_b.reshape(1, Dout))


def _prep_body(degp_ref, x_ref, dinv_ref, diag_ref, hs_ref):
    deg = degp_ref[0, :, 0:1] + degp_ref[1, :, 0:1]
    pos = deg > 0.0
    dinv = jnp.where(pos, lax.rsqrt(jnp.maximum(deg, 1.0)), 0.0)
    dinv_ref[...] = dinv
    diag_ref[...] = jnp.where(pos, 0.0, -1.0)
    hs_ref[...] = dinv * x_ref[...]


def _tc_prep(degp, x, bn=1024):
    N, F = x.shape
    grid = (N // bn,)
    return pl.pallas_call(
        _prep_body,
        grid=grid,
        in_specs=[
            pl.BlockSpec((NC, bn, 8), lambda i: (0, i, 0)),
            pl.BlockSpec((bn, F), lambda i: (i, 0)),
        ],
        out_specs=[
            pl.BlockSpec((bn, 1), lambda i: (i, 0)),
            pl.BlockSpec((bn, 1), lambda i: (i, 0)),
            pl.BlockSpec((bn, F), lambda i: (i, 0)),
        ],
        out_shape=[
            jax.ShapeDtypeStruct((N, 1), jnp.float32),
            jax.ShapeDtypeStruct((N, 1), jnp.float32),
            jax.ShapeDtypeStruct((N, F), jnp.float32),
        ],
    )(degp, x)


def _comb_first_body(accp_ref, h_ref, dinv_ref, diag_ref, tx_ref, hs_ref):
    full = accp_ref[0] + accp_ref[1]
    p = diag_ref[...] * h_ref[...] - dinv_ref[...] * full
    tx_ref[...] = p
    hs_ref[...] = dinv_ref[...] * p


def _comb_rec_body(accp_ref, h_ref, tm2_ref, dinv_ref, diag_ref, tx_ref, hs_ref):
    full = accp_ref[0] + accp_ref[1]
    p = 2.0 * (diag_ref[...] * h_ref[...] - dinv_ref[...] * full) - tm2_ref[...]
    tx_ref[...] = p
    hs_ref[...] = dinv_ref[...] * p


def _tc_combine(accp, h, tm2, dinv, diag, bn=1024):
    N, F = h.shape
    grid = (N // bn,)
    nf = pl.BlockSpec((bn, F), lambda i: (i, 0))
    n1 = pl.BlockSpec((bn, 1), lambda i: (i, 0))
    accs = pl.BlockSpec((NC, bn, F), lambda i: (0, i, 0))
    if tm2 is None:
        body, ins, specs = _comb_first_body, (accp, h, dinv, diag), [accs, nf, n1, n1]
    else:
        body, ins, specs = (_comb_rec_body, (accp, h, tm2, dinv, diag),
                            [accs, nf, nf, n1, n1])
    return pl.pallas_call(
        body,
        grid=grid,
        in_specs=specs,
        out_specs=[nf, nf],
        out_shape=[
            jax.ShapeDtypeStruct((N, F), jnp.float32),
            jax.ShapeDtypeStruct((N, F), jnp.float32),
        ],
    )(*ins)


def _make_mm_body(K, relu):
    def body(*refs):
        t_refs, w_ref, b_ref, o_ref = refs[:K], refs[K], refs[K + 1], refs[K + 2]
        acc = b_ref[...]
        for k in range(K):
            acc = acc + jnp.dot(
                t_refs[k][...], w_ref[k], preferred_element_type=jnp.float32
            )
        o_ref[...] = jnp.maximum(acc, 0.0) if relu else acc

    return body


def _tc_mm(txs, W, b, relu, bn=1024):
    N, Fin = txs[0].shape
    K, _, Fout = W.shape
    grid = (N // bn,)
    nf = pl.BlockSpec((bn, Fin), lambda i: (i, 0))
    return pl.pallas_call(
        _make_mm_body(K, relu),
        grid=grid,
        in_specs=[nf] * K
        + [
            pl.BlockSpec((K, Fin, Fout), lambda i: (0, 0, 0)),
            pl.BlockSpec((1, Fout), lambda i: (0, 0)),
        ],
        out_specs=pl.BlockSpec((bn, Fout), lambda i: (i, 0)),
        out_shape=jax.ShapeDtypeStruct((N, Fout), jnp.float32),
    )(*txs, W, b.reshape(1, Fout))


def _level(x, up_idx, edge_index, W, b, relu):
    Nin, Fin = x.shape
    N = up_idx.shape[0]
    E = edge_index.shape[1]
    K = W.shape[0]
    src = edge_index[0].reshape(E // 128, 128)
    dst = edge_index[1].reshape(E // 128, 128)

    x = _make_sc_gather(Nin, N, Fin)(x, up_idx)

    zerosF = jnp.zeros((N, Fin), jnp.float32)
    ones8 = jnp.ones((128, 8), jnp.float32)
    zeros8 = jnp.zeros((512, 8), jnp.float32)
    degp, = _make_sc_deg_all(((N, E),), x.shape)(src, ones8, zeros8, x)

    dinv, diag, hs = _tc_prep(degp, x)

    prop = _make_sc_prop(N, E, Fin)
    txs = [x]
    accp = prop(hs, src, dst, zerosF)
    tx, hs = _tc_combine(accp, x, None, dinv, diag)
    txs.append(tx)
    for _ in range(2, K):
        accp = prop(hs, src, dst, zerosF)
        tx, hs = _tc_combine(accp, txs[-1], txs[-2], dinv, diag)
        txs.append(tx)

    return _tc_mm(txs, W, b, relu)


def kernel(data, up_idx_4to3, edge_index_3, up_idx_3to2, edge_index_2,
           up_idx_2to1, edge_index_1, up_idx_1to0, edge_index_0,
           fc_W, fc_b, W1, b1, W2, b2, W3, b3, W4, b4):
    x = _tc_fc(data, fc_W, fc_b)
    x = x.reshape(-1, 32)
    x = _level(x, up_idx_4to3, edge_index_3, W1, b1, True)
    x = _level(x, up_idx_3to2, edge_index_2, W2, b2, True)
    x = _level(x, up_idx_2to1, edge_index_1, W3, b3, True)
    x = _level(x, up_idx_1to0, edge_index_0, W4, b4, False)
    return x


# speedup vs baseline: 45.7177x; 1.4643x over previous
import functools

import jax
import jax.numpy as jnp
from jax import lax
from jax.experimental import pallas as pl
from jax.experimental.pallas import tpu as pltpu
from jax.experimental.pallas import tpu_sc as plsc

NC = 2
NS = 16
NW = NC * NS

_MESH = plsc.VectorSubcoreMesh(core_axis_name="c", subcore_axis_name="s")
_SC_PARAMS = pltpu.CompilerParams(use_tc_tiling_on_sc=False)


_NBR = 11


@functools.lru_cache(maxsize=None)
def _make_sc_gather_deg(Nin, M, F, E):
    mslice = M // NW
    CH = 88
    nbg = mslice // CH
    N = M
    erows = E // 128 // NW
    nwav = erows // _NBR
    nslice = N // NS
    ZC = nslice // 11

    @functools.partial(
        pl.kernel,
        out_type=[jax.ShapeDtypeStruct((M, F), jnp.float32),
                  jax.ShapeDtypeStruct((NC, N, 8), jnp.float32)],
        mesh=_MESH,
        compiler_params=_SC_PARAMS,
        scratch_types=[
            pltpu.VMEM_SHARED((N, 8), jnp.float32),
            pltpu.VMEM((mslice,), jnp.int32),
            pltpu.VMEM((mslice, F), jnp.float32),
            pltpu.VMEM((_NBR, 128), jnp.int32),
            pltpu.VMEM((128, 8), jnp.float32),
            pltpu.VMEM((ZC, 8), jnp.float32),
            pltpu.VMEM((ZC, 8), jnp.float32),
            pltpu.SemaphoreType.DMA,
            pltpu.SemaphoreType.DMA,
        ],
    )
    def k(table, idx, src, ones_h, zeros_h, out, degp,
          acc, idx_v, rows_v, dblk, ones_v, zbuf, obuf, sem, sem_s):
        c = lax.axis_index("c")
        s = lax.axis_index("s")
        wid = s * NC + c

        pltpu.sync_copy(ones_h, ones_v)
        pltpu.sync_copy(zeros_h.at[pl.ds(0, ZC)], zbuf)

        def zb(j, carry):
            pltpu.sync_copy(zbuf, acc.at[pl.ds(s * nslice + j * ZC, ZC)])
            return carry

        lax.fori_loop(0, 11, zb, 0)

        base = wid * mslice
        pltpu.sync_copy(idx.at[pl.ds(base, mslice)], idx_v)
        descs = [
            pltpu.async_copy(
                table.at[idx_v.at[pl.ds(b * CH, CH)]],
                rows_v.at[pl.ds(b * CH, CH)],
                sem,
            )
            for b in range(nbg)
        ]
        for d in descs:
            d.wait()
        pltpu.sync_copy(rows_v, out.at[pl.ds(base, mslice)])

        plsc.subcore_barrier()
        rbase = wid * erows

        def eb(w, carry):
            pltpu.sync_copy(src.at[pl.ds(rbase + w * _NBR, _NBR)], dblk)
            sds = [
                pltpu.async_copy(ones_v, acc.at[dblk.at[b]], sem_s, add=True)
                for b in range(_NBR)
            ]
            for d in sds:
                d.wait()
            return carry

        lax.fori_loop(0, nwav, eb, 0)
        plsc.subcore_barrier()

        def wb(j, carry):
            row = s * nslice + j * ZC
            pltpu.sync_copy(acc.at[pl.ds(row, ZC)], obuf)
            pltpu.sync_copy(obuf, degp.at[c, pl.ds(row, ZC)])
            return carry

        lax.fori_loop(0, 11, wb, 0)

    return k


@functools.lru_cache(maxsize=None)
def _make_sc_prop(N, E, F):
    erows = E // 128 // NW
    nslice = N // NS
    ZC = nslice // 11
    nzc = nslice // ZC

    NBR = None
    for nbr in (11, 8, 4, 2, 1):
        if erows % nbr or (erows // nbr) % 2:
            continue
        tile_words = 2 * nbr * 128 * F + 4 * nbr * 128 + ZC * F + 4096
        if N * F + NS * tile_words <= 1_950_000:
            NBR = nbr
            break
    nwav = erows // NBR
    T = nwav // 2

    @functools.partial(
        pl.kernel,
        out_type=jax.ShapeDtypeStruct((NC, N, F), jnp.float32),
        mesh=_MESH,
        compiler_params=_SC_PARAMS,
        scratch_types=[
            pltpu.VMEM_SHARED((N, F), jnp.float32),
            pltpu.VMEM((NBR, 128), jnp.int32),
            pltpu.VMEM((NBR, 128), jnp.int32),
            pltpu.VMEM((NBR, 128), jnp.int32),
            pltpu.VMEM((NBR, 128), jnp.int32),
            pltpu.VMEM((NBR * 128, F), jnp.float32),
            pltpu.VMEM((NBR * 128, F), jnp.float32),
            pltpu.VMEM((ZC, F), jnp.float32),
            pltpu.SemaphoreType.DMA,
            pltpu.SemaphoreType.DMA,
            pltpu.SemaphoreType.DMA,
            pltpu.SemaphoreType.DMA,
        ],
    )
    def k(hs, srcr, dstr, zeros_h, accp, acc, sblk0, dblk0, sblk1, dblk1,
          rows0, rows1, zbuf, sem_g0, sem_g1, sem_s0, sem_s1):
        c = lax.axis_index("c")
        s = lax.axis_index("s")
        wid = s * NC + c
        pltpu.sync_copy(zeros_h.at[pl.ds(0, ZC)], zbuf)

        def zb(j, carry):
            pltpu.sync_copy(zbuf, acc.at[pl.ds(s * nslice + j * ZC, ZC)])
            return carry

        lax.fori_loop(0, nzc, zb, 0)
        plsc.subcore_barrier()

        rbase = wid * erows
        bufs = ((sblk0, dblk0, rows0, sem_g0, sem_s0),
                (sblk1, dblk1, rows1, sem_g1, sem_s1))

        def load_and_fire(wrow, sblk, dblk, rows, sem_g):
            pltpu.sync_copy(srcr.at[pl.ds(wrow, NBR)], sblk)
            pltpu.sync_copy(dstr.at[pl.ds(wrow, NBR)], dblk)
            for b in range(NBR):
                pltpu.async_copy(hs.at[sblk.at[b]],
                                 rows.at[pl.ds(b * 128, 128)], sem_g)

        load_and_fire(rbase, sblk0, dblk0, rows0, sem_g0)
        load_and_fire(rbase + NBR, sblk1, dblk1, rows1, sem_g1)

        def eb(t, carry):
            for p, (sblk, dblk, rows, sem_g, sem_s) in enumerate(bufs):
                for b in range(NBR):
                    pltpu.make_async_copy(
                        hs.at[sblk.at[b]],
                        rows.at[pl.ds(b * 128, 128)], sem_g).wait()
                sds = [
                    pltpu.async_copy(rows.at[pl.ds(b * 128, 128)],
                                     acc.at[dblk.at[b]], sem_s, add=True)
                    for b in range(NBR)
                ]
                for d in sds:
                    d.wait()

                @pl.when(t < T - 1)
                def _():
                    load_and_fire(rbase + (2 * t + p + 2) * NBR,
                                  sblk, dblk, rows, sem_g)

            return carry

        lax.fori_loop(0, T, eb, 0)
        plsc.subcore_barrier()

        def wb(j, carry):
            row = s * nslice + j * ZC
            pltpu.sync_copy(acc.at[pl.ds(row, ZC)], zbuf)
            pltpu.sync_copy(zbuf, accp.at[c, pl.ds(row, ZC)])
            return carry

        lax.fori_loop(0, nzc, wb, 0)

    return k


def _fc_body(d_ref, w_ref, b_ref, o_ref):
    o_ref[...] = (
        jnp.dot(d_ref[...], w_ref[...], preferred_element_type=jnp.float32)
        + b_ref[...]
    )


def _tc_fc(data, fc_W, fc_b):
    B, Dout = data.shape[0], fc_W.shape[1]
    return pl.pallas_call(
        _fc_body,
        out_shape=jax.ShapeDtypeStruct((B, Dout), jnp.float32),
    )(data, fc_W, fc_b.reshape(1, Dout))


def _prep_body(degp_ref, x_ref, dinv_ref, diag_ref, hs_ref):
    deg = degp_ref[0, :, 0:1] + degp_ref[1, :, 0:1]
    pos = deg > 0.0
    dinv = jnp.where(pos, lax.rsqrt(jnp.maximum(deg, 1.0)), 0.0)
    F = x_ref.shape[1]
    dinvF = jnp.broadcast_to(dinv, (dinv.shape[0], F))
    dinv_ref[...] = dinvF
    diag_ref[...] = jnp.broadcast_to(jnp.where(pos, 0.0, -1.0),
                                     (dinv.shape[0], F))
    hs_ref[...] = dinvF * x_ref[...]


def _tc_prep(degp, x, bn=1024):
    N, F = x.shape
    grid = (N // bn,)
    nf = pl.BlockSpec((bn, F), lambda i: (i, 0))
    return pl.pallas_call(
        _prep_body,
        grid=grid,
        in_specs=[
            pl.BlockSpec((NC, bn, 8), lambda i: (0, i, 0)),
            nf,
        ],
        out_specs=[nf, nf, nf],
        out_shape=[
            jax.ShapeDtypeStruct((N, F), jnp.float32),
            jax.ShapeDtypeStruct((N, F), jnp.float32),
            jax.ShapeDtypeStruct((N, F), jnp.float32),
        ],
    )(degp, x)


def _comb_first_body(accp_ref, h_ref, dinv_ref, diag_ref, tx_ref, hs_ref):
    full = accp_ref[0] + accp_ref[1]
    p = diag_ref[...] * h_ref[...] - dinv_ref[...] * full
    tx_ref[...] = p
    hs_ref[...] = dinv_ref[...] * p


def _comb_rec_body(accp_ref, h_ref, tm2_ref, dinv_ref, diag_ref, tx_ref,
                   hs_ref):
    full = accp_ref[0] + accp_ref[1]
    p = (2.0 * (diag_ref[...] * h_ref[...] - dinv_ref[...] * full)
         - tm2_ref[...])
    tx_ref[...] = p
    hs_ref[...] = dinv_ref[...] * p


def _tc_combine(accp, h, tm2, dinvF, diagF, bn=176, W=2048):
    N, F = h.shape
    R = N * F // W
    wide = lambda a: a.reshape(R, W)
    grid = (R // bn,)
    nf = pl.BlockSpec((bn, W), lambda i: (i, 0))
    accs = pl.BlockSpec((NC, bn, W), lambda i: (0, i, 0))
    if tm2 is None:
        body = _comb_first_body
        ins = (accp.reshape(NC, R, W), wide(h), wide(dinvF), wide(diagF))
        specs = [accs, nf, nf, nf]
    else:
        body = _comb_rec_body
        ins = (accp.reshape(NC, R, W), wide(h), wide(tm2), wide(dinvF),
               wide(diagF))
        specs = [accs, nf, nf, nf, nf]
    tx, hs = pl.pallas_call(
        body,
        grid=grid,
        in_specs=specs,
        out_specs=[nf, nf],
        out_shape=[
            jax.ShapeDtypeStruct((R, W), jnp.float32),
            jax.ShapeDtypeStruct((R, W), jnp.float32),
        ],
    )(*ins)
    return tx.reshape(N, F), hs.reshape(N, F)


def _make_comb_mm_body(K, relu):
    def body(*refs):
        (accp_ref, h_ref, tm2_ref, dinv_ref, diag_ref) = refs[:5]
        t_refs = refs[5:5 + K - 2]
        w_ref, b_ref, o_ref = refs[5 + K - 2:]
        full = accp_ref[0] + accp_ref[1]
        txK = (2.0 * (diag_ref[...] * h_ref[...] - dinv_ref[...] * full)
               - tm2_ref[...])
        acc = b_ref[...]
        for k in range(K - 2):
            acc = acc + jnp.dot(t_refs[k][...], w_ref[k],
                                preferred_element_type=jnp.float32)
        acc = acc + jnp.dot(tm2_ref[...], w_ref[K - 2],
                            preferred_element_type=jnp.float32)
        acc = acc + jnp.dot(h_ref[...], w_ref[K - 1],
                            preferred_element_type=jnp.float32)
        acc = acc + jnp.dot(txK, w_ref[K],
                            preferred_element_type=jnp.float32)
        o_ref[...] = jnp.maximum(acc, 0.0) if relu else acc

    return body


def _tc_comb_mm(accp, txs, dinvF, diagF, W, b, relu, bn=1024):
    N, Fin = txs[0].shape
    K, _, Fout = W.shape
    grid = (N // bn,)
    nf = pl.BlockSpec((bn, Fin), lambda i: (i, 0))
    ins = ([accp, txs[-1], txs[-2], dinvF, diagF]
           + list(txs[:-2]) + [W, b.reshape(1, Fout)])
    specs = ([pl.BlockSpec((NC, bn, Fin), lambda i: (0, i, 0)), nf, nf,
              nf, nf] + [nf] * len(txs[:-2])
             + [pl.BlockSpec((K, Fin, Fout), lambda i: (0, 0, 0)),
                pl.BlockSpec((1, Fout), lambda i: (0, 0))])
    return pl.pallas_call(
        _make_comb_mm_body(K - 1, relu),
        grid=grid,
        in_specs=specs,
        out_specs=pl.BlockSpec((bn, Fout), lambda i: (i, 0)),
        out_shape=jax.ShapeDtypeStruct((N, Fout), jnp.float32),
    )(*ins)


def _level(x, up_idx, edge_index, W, b, relu):
    Nin, Fin = x.shape
    N = up_idx.shape[0]
    E = edge_index.shape[1]
    K = W.shape[0]
    src = edge_index[0].reshape(E // 128, 128)
    dst = edge_index[1].reshape(E // 128, 128)

    zerosF = jnp.zeros((N, Fin), jnp.float32)
    ones8 = jnp.ones((128, 8), jnp.float32)
    zeros8 = jnp.zeros((512, 8), jnp.float32)
    x, degp = _make_sc_gather_deg(Nin, N, Fin, E)(x, up_idx, src, ones8,
                                                  zeros8)

    dinvF, diagF, hs = _tc_prep(degp, x)

    prop = _make_sc_prop(N, E, Fin)
    txs = [x]
    accp = prop(hs, src, dst, zerosF)
    tx, hs = _tc_combine(accp, x, None, dinvF, diagF)
    txs.append(tx)
    for _ in range(2, K - 1):
        accp = prop(hs, src, dst, zerosF)
        tx, hs = _tc_combine(accp, txs[-1], txs[-2], dinvF, diagF)
        txs.append(tx)

    accp = prop(hs, src, dst, zerosF)
    return _tc_comb_mm(accp, txs, dinvF, diagF, W, b, relu)


def kernel(data, up_idx_4to3, edge_index_3, up_idx_3to2, edge_index_2,
           up_idx_2to1, edge_index_1, up_idx_1to0, edge_index_0,
           fc_W, fc_b, W1, b1, W2, b2, W3, b3, W4, b4):
    x = _tc_fc(data, fc_W, fc_b)
    x = x.reshape(-1, 32)
    x = _level(x, up_idx_4to3, edge_index_3, W1, b1, True)
    x = _level(x, up_idx_3to2, edge_index_2, W2, b2, True)
    x = _level(x, up_idx_2to1, edge_index_1, W3, b3, True)
    x = _level(x, up_idx_1to0, edge_index_0, W4, b4, False)
    return x


# speedup vs baseline: 53.8388x; 1.1776x over previous
import functools

import jax
import jax.numpy as jnp
from jax import lax
from jax.experimental import pallas as pl
from jax.experimental.pallas import tpu as pltpu
from jax.experimental.pallas import tpu_sc as plsc

NC = 2
NS = 16
NW = NC * NS

_MESH = plsc.VectorSubcoreMesh(core_axis_name="c", subcore_axis_name="s")
_SC_PARAMS = pltpu.CompilerParams(use_tc_tiling_on_sc=False)


_NBR = 11


@functools.lru_cache(maxsize=None)
def _make_sc_gather_deg(Nin, M, F, E):
    mslice = M // NW
    CH = 88
    nbg = mslice // CH
    N = M
    erows = E // 128 // NW
    nwav = erows // _NBR
    nslice = N // NS
    ZC = nslice // 11

    @functools.partial(
        pl.kernel,
        out_type=[jax.ShapeDtypeStruct((M, F), jnp.float32),
                  jax.ShapeDtypeStruct((NC, N, 8), jnp.float32)],
        mesh=_MESH,
        compiler_params=_SC_PARAMS,
        scratch_types=[
            pltpu.VMEM_SHARED((N, 8), jnp.float32),
            pltpu.VMEM((mslice,), jnp.int32),
            pltpu.VMEM((mslice, F), jnp.float32),
            pltpu.VMEM((_NBR, 128), jnp.int32),
            pltpu.VMEM((128, 8), jnp.float32),
            pltpu.SemaphoreType.DMA,
            pltpu.SemaphoreType.DMA,
        ],
    )
    def k(table, idx, src, ones_h, zeros_h, out, degp,
          acc, idx_v, rows_v, dblk, ones_v, sem, sem_s):
        c = lax.axis_index("c")
        s = lax.axis_index("s")
        wid = s * NC + c
        row0 = s * nslice

        pltpu.sync_copy(ones_h, ones_v)
        pltpu.sync_copy(zeros_h.at[pl.ds(row0, nslice)],
                        acc.at[pl.ds(row0, nslice)])

        base = wid * mslice
        pltpu.sync_copy(idx.at[pl.ds(base, mslice)], idx_v)
        descs = [
            pltpu.async_copy(
                table.at[idx_v.at[pl.ds(b * CH, CH)]],
                rows_v.at[pl.ds(b * CH, CH)],
                sem,
            )
            for b in range(nbg)
        ]
        for d in descs:
            d.wait()
        pltpu.sync_copy(rows_v, out.at[pl.ds(base, mslice)])

        plsc.subcore_barrier()
        rbase = wid * erows

        def eb(w, carry):
            pltpu.sync_copy(src.at[pl.ds(rbase + w * _NBR, _NBR)], dblk)
            sds = [
                pltpu.async_copy(ones_v, acc.at[dblk.at[b]], sem_s, add=True)
                for b in range(_NBR)
            ]
            for d in sds:
                d.wait()
            return carry

        lax.fori_loop(0, nwav, eb, 0)
        plsc.subcore_barrier()
        pltpu.sync_copy(acc.at[pl.ds(row0, nslice)],
                        degp.at[c, pl.ds(row0, nslice)])

    return k


@functools.lru_cache(maxsize=None)
def _make_sc_prop(N, E, F):
    erows = E // 128 // NW
    nslice = N // NS

    NBR = None
    for nbr in (11, 8, 4, 2, 1):
        if erows % nbr or (erows // nbr) % 2:
            continue
        tile_words = 2 * nbr * 128 * F + 4 * nbr * 128
        if N * F + NS * tile_words <= 2_050_000:
            NBR = nbr
            break
    nwav = erows // NBR
    T = nwav // 2

    @functools.partial(
        pl.kernel,
        out_type=jax.ShapeDtypeStruct((NC, N, F), jnp.float32),
        mesh=_MESH,
        compiler_params=_SC_PARAMS,
        scratch_types=[
            pltpu.VMEM_SHARED((N, F), jnp.float32),
            pltpu.VMEM((NBR, 128), jnp.int32),
            pltpu.VMEM((NBR, 128), jnp.int32),
            pltpu.VMEM((NBR, 128), jnp.int32),
            pltpu.VMEM((NBR, 128), jnp.int32),
            pltpu.VMEM((NBR * 128, F), jnp.float32),
            pltpu.VMEM((NBR * 128, F), jnp.float32),
            pltpu.SemaphoreType.DMA,
            pltpu.SemaphoreType.DMA,
            pltpu.SemaphoreType.DMA,
            pltpu.SemaphoreType.DMA,
        ],
    )
    def k(hs, srcr, dstr, zeros_h, accp, acc, sblk0, dblk0, sblk1, dblk1,
          rows0, rows1, sem_g0, sem_g1, sem_s0, sem_s1):
        c = lax.axis_index("c")
        s = lax.axis_index("s")
        wid = s * NC + c
        row0 = s * nslice
        pltpu.sync_copy(zeros_h.at[pl.ds(row0, nslice)],
                        acc.at[pl.ds(row0, nslice)])
        plsc.subcore_barrier()

        rbase = wid * erows
        bufs = ((sblk0, dblk0, rows0, sem_g0, sem_s0),
                (sblk1, dblk1, rows1, sem_g1, sem_s1))

        def fire_gathers(sblk, rows, sem_g):
            for b in range(NBR):
                pltpu.async_copy(hs.at[sblk.at[b]],
                                 rows.at[pl.ds(b * 128, 128)], sem_g)

        for p, (sblk, dblk, rows, sem_g, _sem_s) in enumerate(bufs):
            pltpu.sync_copy(srcr.at[pl.ds(rbase + p * NBR, NBR)], sblk)
            pltpu.sync_copy(dstr.at[pl.ds(rbase + p * NBR, NBR)], dblk)
            fire_gathers(sblk, rows, sem_g)

        def eb(t, carry):
            for p, (sblk, dblk, rows, sem_g, sem_s) in enumerate(bufs):
                for b in range(NBR):
                    pltpu.make_async_copy(
                        hs.at[sblk.at[b]],
                        rows.at[pl.ds(b * 128, 128)], sem_g).wait()
                sds = [
                    pltpu.async_copy(rows.at[pl.ds(b * 128, 128)],
                                     acc.at[dblk.at[b]], sem_s, add=True)
                    for b in range(NBR)
                ]
                nrow = rbase + (2 * t + p + 2) * NBR

                @pl.when(t < T - 1)
                def _():
                    pltpu.sync_copy(srcr.at[pl.ds(nrow, NBR)], sblk)

                for d in sds:
                    d.wait()

                @pl.when(t < T - 1)
                def _():
                    pltpu.sync_copy(dstr.at[pl.ds(nrow, NBR)], dblk)
                    fire_gathers(sblk, rows, sem_g)

            return carry

        lax.fori_loop(0, T, eb, 0)
        plsc.subcore_barrier()
        pltpu.sync_copy(acc.at[pl.ds(row0, nslice)],
                        accp.at[c, pl.ds(row0, nslice)])

    return k


def _fc_body(d_ref, w_ref, b_ref, o_ref):
    o_ref[...] = (
        jnp.dot(d_ref[...], w_ref[...], preferred_element_type=jnp.float32)
        + b_ref[...]
    )


def _tc_fc(data, fc_W, fc_b):
    B, Dout = data.shape[0], fc_W.shape[1]
    return pl.pallas_call(
        _fc_body,
        out_shape=jax.ShapeDtypeStruct((B, Dout), jnp.float32),
    )(data, fc_W, fc_b.reshape(1, Dout))


def _prep_body(degp_ref, x_ref, dinv_ref, diag_ref, hs_ref):
    deg = degp_ref[0, :, 0:1] + degp_ref[1, :, 0:1]
    pos = deg > 0.0
    dinv = jnp.where(pos, lax.rsqrt(jnp.maximum(deg, 1.0)), 0.0)
    F = x_ref.shape[1]
    dinvF = jnp.broadcast_to(dinv, (dinv.shape[0], F))
    dinv_ref[...] = dinvF
    diag_ref[...] = jnp.broadcast_to(jnp.where(pos, 0.0, -1.0),
                                     (dinv.shape[0], F))
    hs_ref[...] = dinvF * x_ref[...]


def _tc_prep(degp, x, bn=1024):
    N, F = x.shape
    grid = (N // bn,)
    nf = pl.BlockSpec((bn, F), lambda i: (i, 0))
    return pl.pallas_call(
        _prep_body,
        grid=grid,
        in_specs=[
            pl.BlockSpec((NC, bn, 8), lambda i: (0, i, 0)),
            nf,
        ],
        out_specs=[nf, nf, nf],
        out_shape=[
            jax.ShapeDtypeStruct((N, F), jnp.float32),
            jax.ShapeDtypeStruct((N, F), jnp.float32),
            jax.ShapeDtypeStruct((N, F), jnp.float32),
        ],
    )(degp, x)


def _comb_first_body(accp_ref, h_ref, dinv_ref, diag_ref, tx_ref, hs_ref):
    full = accp_ref[0] + accp_ref[1]
    p = diag_ref[...] * h_ref[...] - dinv_ref[...] * full
    tx_ref[...] = p
    hs_ref[...] = dinv_ref[...] * p


def _comb_rec_body(accp_ref, h_ref, tm2_ref, dinv_ref, diag_ref, tx_ref,
                   hs_ref):
    full = accp_ref[0] + accp_ref[1]
    p = (2.0 * (diag_ref[...] * h_ref[...] - dinv_ref[...] * full)
         - tm2_ref[...])
    tx_ref[...] = p
    hs_ref[...] = dinv_ref[...] * p


def _tc_combine(accp, h, tm2, dinvF, diagF, bn=176, W=2048):
    N, F = h.shape
    R = N * F // W
    wide = lambda a: a.reshape(R, W)
    grid = (R // bn,)
    nf = pl.BlockSpec((bn, W), lambda i: (i, 0))
    accs = pl.BlockSpec((NC, bn, W), lambda i: (0, i, 0))
    if tm2 is None:
        body = _comb_first_body
        ins = (accp.reshape(NC, R, W), wide(h), wide(dinvF), wide(diagF))
        specs = [accs, nf, nf, nf]
    else:
        body = _comb_rec_body
        ins = (accp.reshape(NC, R, W), wide(h), wide(tm2), wide(dinvF),
               wide(diagF))
        specs = [accs, nf, nf, nf, nf]
    tx, hs = pl.pallas_call(
        body,
        grid=grid,
        in_specs=specs,
        out_specs=[nf, nf],
        out_shape=[
            jax.ShapeDtypeStruct((R, W), jnp.float32),
            jax.ShapeDtypeStruct((R, W), jnp.float32),
        ],
    )(*ins)
    return tx.reshape(N, F), hs.reshape(N, F)


def _make_comb_mm_body(K, relu):
    def body(*refs):
        (accp_ref, h_ref, tm2_ref, dinv_ref, diag_ref) = refs[:5]
        t_refs = refs[5:5 + K - 2]
        w_ref, b_ref, o_ref = refs[5 + K - 2:]
        full = accp_ref[0] + accp_ref[1]
        txK = (2.0 * (diag_ref[...] * h_ref[...] - dinv_ref[...] * full)
               - tm2_ref[...])
        acc = b_ref[...]
        for k in range(K - 2):
            acc = acc + jnp.dot(t_refs[k][...], w_ref[k],
                                preferred_element_type=jnp.float32)
        acc = acc + jnp.dot(tm2_ref[...], w_ref[K - 2],
                            preferred_element_type=jnp.float32)
        acc = acc + jnp.dot(h_ref[...], w_ref[K - 1],
                            preferred_element_type=jnp.float32)
        acc = acc + jnp.dot(txK, w_ref[K],
                            preferred_element_type=jnp.float32)
        o_ref[...] = jnp.maximum(acc, 0.0) if relu else acc

    return body


def _tc_comb_mm(accp, txs, dinvF, diagF, W, b, relu, bn=1024):
    N, Fin = txs[0].shape
    K, _, Fout = W.shape
    grid = (N // bn,)
    nf = pl.BlockSpec((bn, Fin), lambda i: (i, 0))
    ins = ([accp, txs[-1], txs[-2], dinvF, diagF]
           + list(txs[:-2]) + [W, b.reshape(1, Fout)])
    specs = ([pl.BlockSpec((NC, bn, Fin), lambda i: (0, i, 0)), nf, nf,
              nf, nf] + [nf] * len(txs[:-2])
             + [pl.BlockSpec((K, Fin, Fout), lambda i: (0, 0, 0)),
                pl.BlockSpec((1, Fout), lambda i: (0, 0))])
    return pl.pallas_call(
        _make_comb_mm_body(K - 1, relu),
        grid=grid,
        in_specs=specs,
        out_specs=pl.BlockSpec((bn, Fout), lambda i: (i, 0)),
        out_shape=jax.ShapeDtypeStruct((N, Fout), jnp.float32),
    )(*ins)


def _level(x, up_idx, edge_index, W, b, relu):
    Nin, Fin = x.shape
    N = up_idx.shape[0]
    E = edge_index.shape[1]
    K = W.shape[0]
    src = edge_index[0].reshape(E // 128, 128)
    dst = edge_index[1].reshape(E // 128, 128)

    zerosF = jnp.zeros((N, Fin), jnp.float32)
    ones8 = jnp.ones((128, 8), jnp.float32)
    zeros8 = jnp.zeros((N, 8), jnp.float32)
    x, degp = _make_sc_gather_deg(Nin, N, Fin, E)(x, up_idx, src, ones8,
                                                  zeros8)

    dinvF, diagF, hs = _tc_prep(degp, x)

    prop = _make_sc_prop(N, E, Fin)
    txs = [x]
    accp = prop(hs, src, dst, zerosF)
    tx, hs = _tc_combine(accp, x, None, dinvF, diagF)
    txs.append(tx)
    for _ in range(2, K - 1):
        accp = prop(hs, src, dst, zerosF)
        tx, hs = _tc_combine(accp, txs[-1], txs[-2], dinvF, diagF)
        txs.append(tx)

    accp = prop(hs, src, dst, zerosF)
    return _tc_comb_mm(accp, txs, dinvF, diagF, W, b, relu)


def kernel(data, up_idx_4to3, edge_index_3, up_idx_3to2, edge_index_2,
           up_idx_2to1, edge_index_1, up_idx_1to0, edge_index_0,
           fc_W, fc_b, W1, b1, W2, b2, W3, b3, W4, b4):
    x = _tc_fc(data, fc_W, fc_b)
    x = x.reshape(-1, 32)
    x = _level(x, up_idx_4to3, edge_index_3, W1, b1, True)
    x = _level(x, up_idx_3to2, edge_index_2, W2, b2, True)
    x = _level(x, up_idx_2to1, edge_index_1, W3, b3, True)
    x = _level(x, up_idx_1to0, edge_index_0, W4, b4, False)
    return x


# speedup vs baseline: 54.2011x; 1.0067x over previous
import functools

import jax
import jax.numpy as jnp
from jax import lax
from jax.experimental import pallas as pl
from jax.experimental.pallas import tpu as pltpu
from jax.experimental.pallas import tpu_sc as plsc

NC = 2
NS = 16
NW = NC * NS

_MESH = plsc.VectorSubcoreMesh(core_axis_name="c", subcore_axis_name="s")
_SC_PARAMS = pltpu.CompilerParams(use_tc_tiling_on_sc=False)


_NBR = 11


@functools.lru_cache(maxsize=None)
def _make_sc_gather_deg(Nin, M, F, E):
    mslice = M // NW
    CH = 88
    nbg = mslice // CH
    N = M
    erows = E // 128 // NW
    NBD = 22
    TD = erows // NBD // 2
    nslice = N // NS

    @functools.partial(
        pl.kernel,
        out_type=[jax.ShapeDtypeStruct((M, F), jnp.float32),
                  jax.ShapeDtypeStruct((NC, N, 8), jnp.float32)],
        mesh=_MESH,
        compiler_params=_SC_PARAMS,
        scratch_types=[
            pltpu.VMEM_SHARED((N, 8), jnp.float32),
            pltpu.VMEM((mslice,), jnp.int32),
            pltpu.VMEM((mslice, F), jnp.float32),
            pltpu.VMEM((NBD, 128), jnp.int32),
            pltpu.VMEM((NBD, 128), jnp.int32),
            pltpu.VMEM((128, 8), jnp.float32),
            pltpu.SemaphoreType.DMA,
            pltpu.SemaphoreType.DMA,
            pltpu.SemaphoreType.DMA,
        ],
    )
    def k(table, idx, src, ones_h, zeros_h, out, degp,
          acc, idx_v, rows_v, dblk0, dblk1, ones_v, sem, sem_s0, sem_s1):
        c = lax.axis_index("c")
        s = lax.axis_index("s")
        wid = s * NC + c
        row0 = s * nslice

        pltpu.sync_copy(ones_h, ones_v)
        pltpu.sync_copy(zeros_h.at[pl.ds(row0, nslice)],
                        acc.at[pl.ds(row0, nslice)])

        base = wid * mslice
        pltpu.sync_copy(idx.at[pl.ds(base, mslice)], idx_v)
        descs = [
            pltpu.async_copy(
                table.at[idx_v.at[pl.ds(b * CH, CH)]],
                rows_v.at[pl.ds(b * CH, CH)],
                sem,
            )
            for b in range(nbg)
        ]
        for d in descs:
            d.wait()
        pltpu.sync_copy(rows_v, out.at[pl.ds(base, mslice)])

        plsc.subcore_barrier()
        rbase = wid * erows
        bufs = ((dblk0, sem_s0), (dblk1, sem_s1))

        pltpu.sync_copy(src.at[pl.ds(rbase, NBD)], dblk0)
        pltpu.sync_copy(src.at[pl.ds(rbase + NBD, NBD)], dblk1)

        def eb(t, carry):
            pends = []
            for p, (dblk, sem_s) in enumerate(bufs):
                pends.append([
                    pltpu.async_copy(ones_v, acc.at[dblk.at[b]], sem_s,
                                     add=True)
                    for b in range(NBD)
                ])
            for p, (dblk, sem_s) in enumerate(bufs):
                for d in pends[p]:
                    d.wait()

                @pl.when(t < TD - 1)
                def _():
                    pltpu.sync_copy(
                        src.at[pl.ds(rbase + (2 * t + p + 2) * NBD, NBD)],
                        dblk)

            return carry

        lax.fori_loop(0, TD, eb, 0)
        plsc.subcore_barrier()
        pltpu.sync_copy(acc.at[pl.ds(row0, nslice)],
                        degp.at[c, pl.ds(row0, nslice)])

    return k


@functools.lru_cache(maxsize=None)
def _make_sc_prop(N, E, F):
    erows = E // 128 // NW
    nslice = N // NS

    NBR = None
    for nbr in (11, 8, 4, 2, 1):
        if erows % nbr or (erows // nbr) % 2:
            continue
        tile_words = 2 * nbr * 128 * F + 4 * nbr * 128
        if N * F + NS * tile_words <= 2_050_000:
            NBR = nbr
            break
    nwav = erows // NBR
    T = nwav // 2

    @functools.partial(
        pl.kernel,
        out_type=jax.ShapeDtypeStruct((NC, N, F), jnp.float32),
        mesh=_MESH,
        compiler_params=_SC_PARAMS,
        scratch_types=[
            pltpu.VMEM_SHARED((N, F), jnp.float32),
            pltpu.VMEM((NBR, 128), jnp.int32),
            pltpu.VMEM((NBR, 128), jnp.int32),
            pltpu.VMEM((NBR, 128), jnp.int32),
            pltpu.VMEM((NBR, 128), jnp.int32),
            pltpu.VMEM((NBR * 128, F), jnp.float32),
            pltpu.VMEM((NBR * 128, F), jnp.float32),
            pltpu.SemaphoreType.DMA,
            pltpu.SemaphoreType.DMA,
            pltpu.SemaphoreType.DMA,
            pltpu.SemaphoreType.DMA,
        ],
    )
    def k(hs, srcr, dstr, zeros_h, accp, acc, sblk0, dblk0, sblk1, dblk1,
          rows0, rows1, sem_g0, sem_g1, sem_s0, sem_s1):
        c = lax.axis_index("c")
        s = lax.axis_index("s")
        wid = s * NC + c
        row0 = s * nslice
        pltpu.sync_copy(zeros_h.at[pl.ds(row0, nslice)],
                        acc.at[pl.ds(row0, nslice)])
        plsc.subcore_barrier()

        rbase = wid * erows
        bufs = ((sblk0, dblk0, rows0, sem_g0, sem_s0),
                (sblk1, dblk1, rows1, sem_g1, sem_s1))

        def fire_gathers(sblk, rows, sem_g):
            for b in range(NBR):
                pltpu.async_copy(hs.at[sblk.at[b]],
                                 rows.at[pl.ds(b * 128, 128)], sem_g)

        for p, (sblk, dblk, rows, sem_g, _sem_s) in enumerate(bufs):
            pltpu.sync_copy(srcr.at[pl.ds(rbase + p * NBR, NBR)], sblk)
            pltpu.sync_copy(dstr.at[pl.ds(rbase + p * NBR, NBR)], dblk)
            fire_gathers(sblk, rows, sem_g)

        def eb(t, carry):
            for p, (sblk, dblk, rows, sem_g, sem_s) in enumerate(bufs):
                for b in range(NBR):
                    pltpu.make_async_copy(
                        hs.at[sblk.at[b]],
                        rows.at[pl.ds(b * 128, 128)], sem_g).wait()
                sds = [
                    pltpu.async_copy(rows.at[pl.ds(b * 128, 128)],
                                     acc.at[dblk.at[b]], sem_s, add=True)
                    for b in range(NBR)
                ]
                nrow = rbase + (2 * t + p + 2) * NBR

                @pl.when(t < T - 1)
                def _():
                    pltpu.sync_copy(srcr.at[pl.ds(nrow, NBR)], sblk)

                for d in sds:
                    d.wait()

                @pl.when(t < T - 1)
                def _():
                    pltpu.sync_copy(dstr.at[pl.ds(nrow, NBR)], dblk)
                    fire_gathers(sblk, rows, sem_g)

            return carry

        lax.fori_loop(0, T, eb, 0)
        plsc.subcore_barrier()
        pltpu.sync_copy(acc.at[pl.ds(row0, nslice)],
                        accp.at[c, pl.ds(row0, nslice)])

    return k


def _fc_body(d_ref, w_ref, b_ref, o_ref):
    o_ref[...] = (
        jnp.dot(d_ref[...], w_ref[...], preferred_element_type=jnp.float32)
        + b_ref[...]
    )


def _tc_fc(data, fc_W, fc_b):
    B, Dout = data.shape[0], fc_W.shape[1]
    return pl.pallas_call(
        _fc_body,
        out_shape=jax.ShapeDtypeStruct((B, Dout), jnp.float32),
    )(data, fc_W, fc_b.reshape(1, Dout))


def _prep_body(degp_ref, x_ref, dinv_ref, diag_ref, hs_ref):
    deg = degp_ref[0, :, 0:1] + degp_ref[1, :, 0:1]
    pos = deg > 0.0
    dinv = jnp.where(pos, lax.rsqrt(jnp.maximum(deg, 1.0)), 0.0)
    F = x_ref.shape[1]
    dinvF = jnp.broadcast_to(dinv, (dinv.shape[0], F))
    dinv_ref[...] = dinvF
    diag_ref[...] = jnp.broadcast_to(jnp.where(pos, 0.0, -1.0),
                                     (dinv.shape[0], F))
    hs_ref[...] = dinvF * x_ref[...]


def _tc_prep(degp, x, bn=1024):
    N, F = x.shape
    grid = (N // bn,)
    nf = pl.BlockSpec((bn, F), lambda i: (i, 0))
    return pl.pallas_call(
        _prep_body,
        grid=grid,
        in_specs=[
            pl.BlockSpec((NC, bn, 8), lambda i: (0, i, 0)),
            nf,
        ],
        out_specs=[nf, nf, nf],
        out_shape=[
            jax.ShapeDtypeStruct((N, F), jnp.float32),
            jax.ShapeDtypeStruct((N, F), jnp.float32),
            jax.ShapeDtypeStruct((N, F), jnp.float32),
        ],
    )(degp, x)


def _comb_first_body(accp_ref, h_ref, dinv_ref, diag_ref, tx_ref, hs_ref):
    full = accp_ref[0] + accp_ref[1]
    p = diag_ref[...] * h_ref[...] - dinv_ref[...] * full
    tx_ref[...] = p
    hs_ref[...] = dinv_ref[...] * p


def _comb_rec_body(accp_ref, h_ref, tm2_ref, dinv_ref, diag_ref, tx_ref,
                   hs_ref):
    full = accp_ref[0] + accp_ref[1]
    p = (2.0 * (diag_ref[...] * h_ref[...] - dinv_ref[...] * full)
         - tm2_ref[...])
    tx_ref[...] = p
    hs_ref[...] = dinv_ref[...] * p


def _tc_combine(accp, h, tm2, dinvF, diagF, bn=176, W=2048):
    N, F = h.shape
    R = N * F // W
    wide = lambda a: a.reshape(R, W)
    grid = (R // bn,)
    nf = pl.BlockSpec((bn, W), lambda i: (i, 0))
    accs = pl.BlockSpec((NC, bn, W), lambda i: (0, i, 0))
    if tm2 is None:
        body = _comb_first_body
        ins = (accp.reshape(NC, R, W), wide(h), wide(dinvF), wide(diagF))
        specs = [accs, nf, nf, nf]
    else:
        body = _comb_rec_body
        ins = (accp.reshape(NC, R, W), wide(h), wide(tm2), wide(dinvF),
               wide(diagF))
        specs = [accs, nf, nf, nf, nf]
    tx, hs = pl.pallas_call(
        body,
        grid=grid,
        in_specs=specs,
        out_specs=[nf, nf],
        out_shape=[
            jax.ShapeDtypeStruct((R, W), jnp.float32),
            jax.ShapeDtypeStruct((R, W), jnp.float32),
        ],
    )(*ins)
    return tx.reshape(N, F), hs.reshape(N, F)


def _make_comb_mm_body(K, relu):
    def body(*refs):
        (accp_ref, h_ref, tm2_ref, dinv_ref, diag_ref) = refs[:5]
        t_refs = refs[5:5 + K - 2]
        w_ref, b_ref, o_ref = refs[5 + K - 2:]
        full = accp_ref[0] + accp_ref[1]
        txK = (2.0 * (diag_ref[...] * h_ref[...] - dinv_ref[...] * full)
               - tm2_ref[...])
        acc = b_ref[...]
        for k in range(K - 2):
            acc = acc + jnp.dot(t_refs[k][...], w_ref[k],
                                preferred_element_type=jnp.float32)
        acc = acc + jnp.dot(tm2_ref[...], w_ref[K - 2],
                            preferred_element_type=jnp.float32)
        acc = acc + jnp.dot(h_ref[...], w_ref[K - 1],
                            preferred_element_type=jnp.float32)
        acc = acc + jnp.dot(txK, w_ref[K],
                            preferred_element_type=jnp.float32)
        o_ref[...] = jnp.maximum(acc, 0.0) if relu else acc

    return body


def _tc_comb_mm(accp, txs, dinvF, diagF, W, b, relu, bn=1024):
    N, Fin = txs[0].shape
    K, _, Fout = W.shape
    grid = (N // bn,)
    nf = pl.BlockSpec((bn, Fin), lambda i: (i, 0))
    ins = ([accp, txs[-1], txs[-2], dinvF, diagF]
           + list(txs[:-2]) + [W, b.reshape(1, Fout)])
    specs = ([pl.BlockSpec((NC, bn, Fin), lambda i: (0, i, 0)), nf, nf,
              nf, nf] + [nf] * len(txs[:-2])
             + [pl.BlockSpec((K, Fin, Fout), lambda i: (0, 0, 0)),
                pl.BlockSpec((1, Fout), lambda i: (0, 0))])
    return pl.pallas_call(
        _make_comb_mm_body(K - 1, relu),
        grid=grid,
        in_specs=specs,
        out_specs=pl.BlockSpec((bn, Fout), lambda i: (i, 0)),
        out_shape=jax.ShapeDtypeStruct((N, Fout), jnp.float32),
    )(*ins)


def _level(x, up_idx, edge_index, W, b, relu):
    Nin, Fin = x.shape
    N = up_idx.shape[0]
    E = edge_index.shape[1]
    K = W.shape[0]
    src = edge_index[0].reshape(E // 128, 128)
    dst = edge_index[1].reshape(E // 128, 128)

    zerosF = jnp.zeros((N, Fin), jnp.float32)
    ones8 = jnp.ones((128, 8), jnp.float32)
    zeros8 = jnp.zeros((N, 8), jnp.float32)
    x, degp = _make_sc_gather_deg(Nin, N, Fin, E)(x, up_idx, src, ones8,
                                                  zeros8)

    dinvF, diagF, hs = _tc_prep(degp, x)

    prop = _make_sc_prop(N, E, Fin)
    txs = [x]
    accp = prop(hs, src, dst, zerosF)
    tx, hs = _tc_combine(accp, x, None, dinvF, diagF)
    txs.append(tx)
    for _ in range(2, K - 1):
        accp = prop(hs, src, dst, zerosF)
        tx, hs = _tc_combine(accp, txs[-1], txs[-2], dinvF, diagF)
        txs.append(tx)

    accp = prop(hs, src, dst, zerosF)
    return _tc_comb_mm(accp, txs, dinvF, diagF, W, b, relu)


def kernel(data, up_idx_4to3, edge_index_3, up_idx_3to2, edge_index_2,
           up_idx_2to1, edge_index_1, up_idx_1to0, edge_index_0,
           fc_W, fc_b, W1, b1, W2, b2, W3, b3, W4, b4):
    x = _tc_fc(data, fc_W, fc_b)
    x = x.reshape(-1, 32)
    x = _level(x, up_idx_4to3, edge_index_3, W1, b1, True)
    x = _level(x, up_idx_3to2, edge_index_2, W2, b2, True)
    x = _level(x, up_idx_2to1, edge_index_1, W3, b3, True)
    x = _level(x, up_idx_1to0, edge_index_0, W4, b4, False)
    return x


# speedup vs baseline: 54.2110x; 1.0002x over previous
import functools

import jax
import jax.numpy as jnp
from jax import lax
from jax.experimental import pallas as pl
from jax.experimental.pallas import tpu as pltpu
from jax.experimental.pallas import tpu_sc as plsc

NC = 2
NS = 16
NW = NC * NS

_MESH = plsc.VectorSubcoreMesh(core_axis_name="c", subcore_axis_name="s")
_SC_PARAMS = pltpu.CompilerParams(use_tc_tiling_on_sc=False,
                                  skip_device_barrier=True)


_NBR = 11


@functools.lru_cache(maxsize=None)
def _make_sc_gather_deg(Nin, M, F, E):
    mslice = M // NW
    CH = 88
    nbg = mslice // CH
    N = M
    erows = E // 128 // NW
    NBD = 22
    TD = erows // NBD // 2
    nslice = N // NS

    @functools.partial(
        pl.kernel,
        out_type=[jax.ShapeDtypeStruct((M, F), jnp.float32),
                  jax.ShapeDtypeStruct((NC, N, 8), jnp.float32)],
        mesh=_MESH,
        compiler_params=_SC_PARAMS,
        scratch_types=[
            pltpu.VMEM_SHARED((N, 8), jnp.float32),
            pltpu.VMEM((mslice,), jnp.int32),
            pltpu.VMEM((mslice, F), jnp.float32),
            pltpu.VMEM((NBD, 128), jnp.int32),
            pltpu.VMEM((NBD, 128), jnp.int32),
            pltpu.VMEM((128, 8), jnp.float32),
            pltpu.SemaphoreType.DMA,
            pltpu.SemaphoreType.DMA,
            pltpu.SemaphoreType.DMA,
        ],
    )
    def k(table, idx, src, ones_h, zeros_h, out, degp,
          acc, idx_v, rows_v, dblk0, dblk1, ones_v, sem, sem_s0, sem_s1):
        c = lax.axis_index("c")
        s = lax.axis_index("s")
        wid = s * NC + c
        row0 = s * nslice

        pltpu.sync_copy(ones_h, ones_v)
        pltpu.sync_copy(zeros_h.at[pl.ds(row0, nslice)],
                        acc.at[pl.ds(row0, nslice)])

        base = wid * mslice
        pltpu.sync_copy(idx.at[pl.ds(base, mslice)], idx_v)
        descs = [
            pltpu.async_copy(
                table.at[idx_v.at[pl.ds(b * CH, CH)]],
                rows_v.at[pl.ds(b * CH, CH)],
                sem,
            )
            for b in range(nbg)
        ]
        for d in descs:
            d.wait()
        pltpu.sync_copy(rows_v, out.at[pl.ds(base, mslice)])

        plsc.subcore_barrier()
        rbase = wid * erows
        bufs = ((dblk0, sem_s0), (dblk1, sem_s1))

        pltpu.sync_copy(src.at[pl.ds(rbase, NBD)], dblk0)
        pltpu.sync_copy(src.at[pl.ds(rbase + NBD, NBD)], dblk1)

        def eb(t, carry):
            pends = []
            for p, (dblk, sem_s) in enumerate(bufs):
                pends.append([
                    pltpu.async_copy(ones_v, acc.at[dblk.at[b]], sem_s,
                                     add=True)
                    for b in range(NBD)
                ])
            for p, (dblk, sem_s) in enumerate(bufs):
                for d in pends[p]:
                    d.wait()

                @pl.when(t < TD - 1)
                def _():
                    pltpu.sync_copy(
                        src.at[pl.ds(rbase + (2 * t + p + 2) * NBD, NBD)],
                        dblk)

            return carry

        lax.fori_loop(0, TD, eb, 0)
        plsc.subcore_barrier()
        pltpu.sync_copy(acc.at[pl.ds(row0, nslice)],
                        degp.at[c, pl.ds(row0, nslice)])

    return k


@functools.lru_cache(maxsize=None)
def _make_sc_prop(N, E, F):
    erows = E // 128 // NW
    nslice = N // NS

    NBR = None
    for nbr in (11, 8, 4, 2, 1):
        if erows % nbr or (erows // nbr) % 2:
            continue
        tile_words = 2 * nbr * 128 * F + 4 * nbr * 128
        if N * F + NS * tile_words <= 2_050_000:
            NBR = nbr
            break
    nwav = erows // NBR
    T = nwav // 2

    @functools.partial(
        pl.kernel,
        out_type=jax.ShapeDtypeStruct((NC, N, F), jnp.float32),
        mesh=_MESH,
        compiler_params=_SC_PARAMS,
        scratch_types=[
            pltpu.VMEM_SHARED((N, F), jnp.float32),
            pltpu.VMEM((NBR, 128), jnp.int32),
            pltpu.VMEM((NBR, 128), jnp.int32),
            pltpu.VMEM((NBR, 128), jnp.int32),
            pltpu.VMEM((NBR, 128), jnp.int32),
            pltpu.VMEM((NBR * 128, F), jnp.float32),
            pltpu.VMEM((NBR * 128, F), jnp.float32),
            pltpu.SemaphoreType.DMA,
            pltpu.SemaphoreType.DMA,
            pltpu.SemaphoreType.DMA,
            pltpu.SemaphoreType.DMA,
        ],
    )
    def k(hs, srcr, dstr, zeros_h, accp, acc, sblk0, dblk0, sblk1, dblk1,
          rows0, rows1, sem_g0, sem_g1, sem_s0, sem_s1):
        c = lax.axis_index("c")
        s = lax.axis_index("s")
        wid = s * NC + c
        row0 = s * nslice
        pltpu.sync_copy(zeros_h.at[pl.ds(row0, nslice)],
                        acc.at[pl.ds(row0, nslice)])
        plsc.subcore_barrier()

        rbase = wid * erows
        bufs = ((sblk0, dblk0, rows0, sem_g0, sem_s0),
                (sblk1, dblk1, rows1, sem_g1, sem_s1))

        def fire_gathers(sblk, rows, sem_g):
            for b in range(NBR):
                pltpu.async_copy(hs.at[sblk.at[b]],
                                 rows.at[pl.ds(b * 128, 128)], sem_g)

        for p, (sblk, dblk, rows, sem_g, _sem_s) in enumerate(bufs):
            pltpu.sync_copy(srcr.at[pl.ds(rbase + p * NBR, NBR)], sblk)
            pltpu.sync_copy(dstr.at[pl.ds(rbase + p * NBR, NBR)], dblk)
            fire_gathers(sblk, rows, sem_g)

        def eb(t, carry):
            for p, (sblk, dblk, rows, sem_g, sem_s) in enumerate(bufs):
                for b in range(NBR):
                    pltpu.make_async_copy(
                        hs.at[sblk.at[b]],
                        rows.at[pl.ds(b * 128, 128)], sem_g).wait()
                sds = [
                    pltpu.async_copy(rows.at[pl.ds(b * 128, 128)],
                                     acc.at[dblk.at[b]], sem_s, add=True)
                    for b in range(NBR)
                ]
                nrow = rbase + (2 * t + p + 2) * NBR

                @pl.when(t < T - 1)
                def _():
                    pltpu.sync_copy(srcr.at[pl.ds(nrow, NBR)], sblk)

                for d in sds:
                    d.wait()

                @pl.when(t < T - 1)
                def _():
                    pltpu.sync_copy(dstr.at[pl.ds(nrow, NBR)], dblk)
                    fire_gathers(sblk, rows, sem_g)

            return carry

        lax.fori_loop(0, T, eb, 0)
        plsc.subcore_barrier()
        pltpu.sync_copy(acc.at[pl.ds(row0, nslice)],
                        accp.at[c, pl.ds(row0, nslice)])

    return k


def _fc_body(d_ref, w_ref, b_ref, o_ref):
    o_ref[...] = (
        jnp.dot(d_ref[...], w_ref[...], preferred_element_type=jnp.float32)
        + b_ref[...]
    )


def _tc_fc(data, fc_W, fc_b):
    B, Dout = data.shape[0], fc_W.shape[1]
    return pl.pallas_call(
        _fc_body,
        out_shape=jax.ShapeDtypeStruct((B, Dout), jnp.float32),
    )(data, fc_W, fc_b.reshape(1, Dout))


def _prep_body(degp_ref, x_ref, dinv_ref, diag_ref, hs_ref):
    deg = degp_ref[0, :, 0:1] + degp_ref[1, :, 0:1]
    pos = deg > 0.0
    dinv = jnp.where(pos, lax.rsqrt(jnp.maximum(deg, 1.0)), 0.0)
    F = x_ref.shape[1]
    dinvF = jnp.broadcast_to(dinv, (dinv.shape[0], F))
    dinv_ref[...] = dinvF
    diag_ref[...] = jnp.broadcast_to(jnp.where(pos, 0.0, -1.0),
                                     (dinv.shape[0], F))
    hs_ref[...] = dinvF * x_ref[...]


def _tc_prep(degp, x, bn=1024):
    N, F = x.shape
    grid = (N // bn,)
    nf = pl.BlockSpec((bn, F), lambda i: (i, 0))
    return pl.pallas_call(
        _prep_body,
        grid=grid,
        in_specs=[
            pl.BlockSpec((NC, bn, 8), lambda i: (0, i, 0)),
            nf,
        ],
        out_specs=[nf, nf, nf],
        out_shape=[
            jax.ShapeDtypeStruct((N, F), jnp.float32),
            jax.ShapeDtypeStruct((N, F), jnp.float32),
            jax.ShapeDtypeStruct((N, F), jnp.float32),
        ],
    )(degp, x)


def _comb_first_body(accp_ref, h_ref, dinv_ref, diag_ref, tx_ref, hs_ref):
    full = accp_ref[0] + accp_ref[1]
    p = diag_ref[...] * h_ref[...] - dinv_ref[...] * full
    tx_ref[...] = p
    hs_ref[...] = dinv_ref[...] * p


def _comb_rec_body(accp_ref, h_ref, tm2_ref, dinv_ref, diag_ref, tx_ref,
                   hs_ref):
    full = accp_ref[0] + accp_ref[1]
    p = (2.0 * (diag_ref[...] * h_ref[...] - dinv_ref[...] * full)
         - tm2_ref[...])
    tx_ref[...] = p
    hs_ref[...] = dinv_ref[...] * p


def _tc_combine(accp, h, tm2, dinvF, diagF, bn=176, W=2048):
    N, F = h.shape
    R = N * F // W
    wide = lambda a: a.reshape(R, W)
    grid = (R // bn,)
    nf = pl.BlockSpec((bn, W), lambda i: (i, 0))
    accs = pl.BlockSpec((NC, bn, W), lambda i: (0, i, 0))
    if tm2 is None:
        body = _comb_first_body
        ins = (accp.reshape(NC, R, W), wide(h), wide(dinvF), wide(diagF))
        specs = [accs, nf, nf, nf]
    else:
        body = _comb_rec_body
        ins = (accp.reshape(NC, R, W), wide(h), wide(tm2), wide(dinvF),
               wide(diagF))
        specs = [accs, nf, nf, nf, nf]
    tx, hs = pl.pallas_call(
        body,
        grid=grid,
        in_specs=specs,
        out_specs=[nf, nf],
        out_shape=[
            jax.ShapeDtypeStruct((R, W), jnp.float32),
            jax.ShapeDtypeStruct((R, W), jnp.float32),
        ],
    )(*ins)
    return tx.reshape(N, F), hs.reshape(N, F)


def _make_comb_mm_body(K, relu):
    def body(*refs):
        (accp_ref, h_ref, tm2_ref, dinv_ref, diag_ref) = refs[:5]
        t_refs = refs[5:5 + K - 2]
        w_ref, b_ref, o_ref = refs[5 + K - 2:]
        full = accp_ref[0] + accp_ref[1]
        txK = (2.0 * (diag_ref[...] * h_ref[...] - dinv_ref[...] * full)
               - tm2_ref[...])
        acc = b_ref[...]
        for k in range(K - 2):
            acc = acc + jnp.dot(t_refs[k][...], w_ref[k],
                                preferred_element_type=jnp.float32)
        acc = acc + jnp.dot(tm2_ref[...], w_ref[K - 2],
                            preferred_element_type=jnp.float32)
        acc = acc + jnp.dot(h_ref[...], w_ref[K - 1],
                            preferred_element_type=jnp.float32)
        acc = acc + jnp.dot(txK, w_ref[K],
                            preferred_element_type=jnp.float32)
        o_ref[...] = jnp.maximum(acc, 0.0) if relu else acc

    return body


def _tc_comb_mm(accp, txs, dinvF, diagF, W, b, relu, bn=1024):
    N, Fin = txs[0].shape
    K, _, Fout = W.shape
    grid = (N // bn,)
    nf = pl.BlockSpec((bn, Fin), lambda i: (i, 0))
    ins = ([accp, txs[-1], txs[-2], dinvF, diagF]
           + list(txs[:-2]) + [W, b.reshape(1, Fout)])
    specs = ([pl.BlockSpec((NC, bn, Fin), lambda i: (0, i, 0)), nf, nf,
              nf, nf] + [nf] * len(txs[:-2])
             + [pl.BlockSpec((K, Fin, Fout), lambda i: (0, 0, 0)),
                pl.BlockSpec((1, Fout), lambda i: (0, 0))])
    return pl.pallas_call(
        _make_comb_mm_body(K - 1, relu),
        grid=grid,
        in_specs=specs,
        out_specs=pl.BlockSpec((bn, Fout), lambda i: (i, 0)),
        out_shape=jax.ShapeDtypeStruct((N, Fout), jnp.float32),
    )(*ins)


def _level(x, up_idx, edge_index, W, b, relu):
    Nin, Fin = x.shape
    N = up_idx.shape[0]
    E = edge_index.shape[1]
    K = W.shape[0]
    src = edge_index[0].reshape(E // 128, 128)
    dst = edge_index[1].reshape(E // 128, 128)

    zerosF = jnp.zeros((N, Fin), jnp.float32)
    ones8 = jnp.ones((128, 8), jnp.float32)
    zeros8 = jnp.zeros((N, 8), jnp.float32)
    x, degp = _make_sc_gather_deg(Nin, N, Fin, E)(x, up_idx, src, ones8,
                                                  zeros8)

    dinvF, diagF, hs = _tc_prep(degp, x)

    prop = _make_sc_prop(N, E, Fin)
    txs = [x]
    accp = prop(hs, src, dst, zerosF)
    tx, hs = _tc_combine(accp, x, None, dinvF, diagF)
    txs.append(tx)
    for _ in range(2, K - 1):
        accp = prop(hs, src, dst, zerosF)
        tx, hs = _tc_combine(accp, txs[-1], txs[-2], dinvF, diagF)
        txs.append(tx)

    accp = prop(hs, src, dst, zerosF)
    return _tc_comb_mm(accp, txs, dinvF, diagF, W, b, relu)


def kernel(data, up_idx_4to3, edge_index_3, up_idx_3to2, edge_index_2,
           up_idx_2to1, edge_index_1, up_idx_1to0, edge_index_0,
           fc_W, fc_b, W1, b1, W2, b2, W3, b3, W4, b4):
    x = _tc_fc(data, fc_W, fc_b)
    x = x.reshape(-1, 32)
    x = _level(x, up_idx_4to3, edge_index_3, W1, b1, True)
    x = _level(x, up_idx_3to2, edge_index_2, W2, b2, True)
    x = _level(x, up_idx_2to1, edge_index_1, W3, b3, True)
    x = _level(x, up_idx_1to0, edge_index_0, W4, b4, False)
    return x


# speedup vs baseline: 54.2146x; 1.0001x over previous
import functools

import jax
import jax.numpy as jnp
from jax import lax
from jax.experimental import pallas as pl
from jax.experimental.pallas import tpu as pltpu
from jax.experimental.pallas import tpu_sc as plsc

NC = 2
NS = 16
NW = NC * NS

_MESH = plsc.VectorSubcoreMesh(core_axis_name="c", subcore_axis_name="s")
_SC_PARAMS = pltpu.CompilerParams(use_tc_tiling_on_sc=False)


_NBR = 11


@functools.lru_cache(maxsize=None)
def _make_sc_gather_deg(Nin, M, F, E):
    mslice = M // NW
    CH = 88
    nbg = mslice // CH
    N = M
    erows = E // 128 // NW
    NBD = 22
    TD = erows // NBD // 2
    nslice = N // NS

    @functools.partial(
        pl.kernel,
        out_type=[jax.ShapeDtypeStruct((M, F), jnp.float32),
                  jax.ShapeDtypeStruct((NC, N, 8), jnp.float32)],
        mesh=_MESH,
        compiler_params=_SC_PARAMS,
        scratch_types=[
            pltpu.VMEM_SHARED((N, 8), jnp.float32),
            pltpu.VMEM((mslice,), jnp.int32),
            pltpu.VMEM((mslice, F), jnp.float32),
            pltpu.VMEM((NBD, 128), jnp.int32),
            pltpu.VMEM((NBD, 128), jnp.int32),
            pltpu.VMEM((128, 8), jnp.float32),
            pltpu.SemaphoreType.DMA,
            pltpu.SemaphoreType.DMA,
            pltpu.SemaphoreType.DMA,
        ],
    )
    def k(table, idx, src, ones_h, zeros_h, out, degp,
          acc, idx_v, rows_v, dblk0, dblk1, ones_v, sem, sem_s0, sem_s1):
        c = lax.axis_index("c")
        s = lax.axis_index("s")
        wid = s * NC + c
        row0 = s * nslice

        pltpu.sync_copy(ones_h, ones_v)
        pltpu.sync_copy(zeros_h.at[pl.ds(row0, nslice)],
                        acc.at[pl.ds(row0, nslice)])

        base = wid * mslice
        pltpu.sync_copy(idx.at[pl.ds(base, mslice)], idx_v)
        descs = [
            pltpu.async_copy(
                table.at[idx_v.at[pl.ds(b * CH, CH)]],
                rows_v.at[pl.ds(b * CH, CH)],
                sem,
            )
            for b in range(nbg)
        ]
        for d in descs:
            d.wait()
        pltpu.sync_copy(rows_v, out.at[pl.ds(base, mslice)])

        plsc.subcore_barrier()
        rbase = wid * erows
        bufs = ((dblk0, sem_s0), (dblk1, sem_s1))

        pltpu.sync_copy(src.at[pl.ds(rbase, NBD)], dblk0)
        pltpu.sync_copy(src.at[pl.ds(rbase + NBD, NBD)], dblk1)

        def eb(t, carry):
            pends = []
            for p, (dblk, sem_s) in enumerate(bufs):
                pends.append([
                    pltpu.async_copy(ones_v, acc.at[dblk.at[b]], sem_s,
                                     add=True)
                    for b in range(NBD)
                ])
            for p, (dblk, sem_s) in enumerate(bufs):
                for d in pends[p]:
                    d.wait()

                @pl.when(t < TD - 1)
                def _():
                    pltpu.sync_copy(
                        src.at[pl.ds(rbase + (2 * t + p + 2) * NBD, NBD)],
                        dblk)

            return carry

        lax.fori_loop(0, TD, eb, 0)
        plsc.subcore_barrier()
        pltpu.sync_copy(acc.at[pl.ds(row0, nslice)],
                        degp.at[c, pl.ds(row0, nslice)])

    return k


@functools.lru_cache(maxsize=None)
def _make_sc_prop(N, E, F):
    erows = E // 128 // NW
    nslice = N // NS

    NBR = None
    for nbr in (11, 8, 4, 2, 1):
        if erows % nbr or (erows // nbr) % 2:
            continue
        tile_words = 2 * nbr * 128 * F + 4 * nbr * 128
        if N * F + NS * tile_words <= 2_050_000:
            NBR = nbr
            break
    nwav = erows // NBR
    T = nwav // 2

    @functools.partial(
        pl.kernel,
        out_type=jax.ShapeDtypeStruct((NC, N, F), jnp.float32),
        mesh=_MESH,
        compiler_params=_SC_PARAMS,
        scratch_types=[
            pltpu.VMEM_SHARED((N, F), jnp.float32),
            pltpu.VMEM((NBR, 128), jnp.int32),
            pltpu.VMEM((NBR, 128), jnp.int32),
            pltpu.VMEM((NBR, 128), jnp.int32),
            pltpu.VMEM((NBR, 128), jnp.int32),
            pltpu.VMEM((NBR * 128, F), jnp.float32),
            pltpu.VMEM((NBR * 128, F), jnp.float32),
            pltpu.SemaphoreType.DMA,
            pltpu.SemaphoreType.DMA,
            pltpu.SemaphoreType.DMA,
            pltpu.SemaphoreType.DMA,
        ],
    )
    def k(hs, srcr, dstr, zeros_h, accp, acc, sblk0, dblk0, sblk1, dblk1,
          rows0, rows1, sem_g0, sem_g1, sem_s0, sem_s1):
        c = lax.axis_index("c")
        s = lax.axis_index("s")
        wid = s * NC + c
        row0 = s * nslice
        pltpu.sync_copy(zeros_h.at[pl.ds(row0, nslice)],
                        acc.at[pl.ds(row0, nslice)])
        plsc.subcore_barrier()

        rbase = wid * erows
        bufs = ((sblk0, dblk0, rows0, sem_g0, sem_s0),
                (sblk1, dblk1, rows1, sem_g1, sem_s1))

        def fire_gathers(sblk, rows, sem_g):
            for b in range(NBR):
                pltpu.async_copy(hs.at[sblk.at[b]],
                                 rows.at[pl.ds(b * 128, 128)], sem_g)

        for p, (sblk, dblk, rows, sem_g, _sem_s) in enumerate(bufs):
            pltpu.sync_copy(srcr.at[pl.ds(rbase + p * NBR, NBR)], sblk)
            pltpu.sync_copy(dstr.at[pl.ds(rbase + p * NBR, NBR)], dblk)
            fire_gathers(sblk, rows, sem_g)

        def eb(t, carry):
            for p, (sblk, dblk, rows, sem_g, sem_s) in enumerate(bufs):
                for b in range(NBR):
                    pltpu.make_async_copy(
                        hs.at[sblk.at[b]],
                        rows.at[pl.ds(b * 128, 128)], sem_g).wait()
                sds = [
                    pltpu.async_copy(rows.at[pl.ds(b * 128, 128)],
                                     acc.at[dblk.at[b]], sem_s, add=True)
                    for b in range(NBR)
                ]
                nrow = rbase + (2 * t + p + 2) * NBR

                @pl.when(t < T - 1)
                def _():
                    pltpu.sync_copy(srcr.at[pl.ds(nrow, NBR)], sblk)

                for d in sds:
                    d.wait()

                @pl.when(t < T - 1)
                def _():
                    pltpu.sync_copy(dstr.at[pl.ds(nrow, NBR)], dblk)
                    fire_gathers(sblk, rows, sem_g)

            return carry

        lax.fori_loop(0, T, eb, 0)
        plsc.subcore_barrier()
        pltpu.sync_copy(acc.at[pl.ds(row0, nslice)],
                        accp.at[c, pl.ds(row0, nslice)])

    return k


def _fc_body(d_ref, w_ref, b_ref, o_ref):
    o_ref[...] = (
        jnp.dot(d_ref[...], w_ref[...], preferred_element_type=jnp.float32)
        + b_ref[...]
    )


def _tc_fc(data, fc_W, fc_b):
    B, Dout = data.shape[0], fc_W.shape[1]
    return pl.pallas_call(
        _fc_body,
        out_shape=jax.ShapeDtypeStruct((B, Dout), jnp.float32),
    )(data, fc_W, fc_b.reshape(1, Dout))


def _prep_body(degp_ref, x_ref, dinv_ref, diag_ref, hs_ref):
    deg = degp_ref[0, :, 0:1] + degp_ref[1, :, 0:1]
    pos = deg > 0.0
    dinv = jnp.where(pos, lax.rsqrt(jnp.maximum(deg, 1.0)), 0.0)
    F = x_ref.shape[1]
    dinvF = jnp.broadcast_to(dinv, (dinv.shape[0], F))
    dinv_ref[...] = dinvF
    diag_ref[...] = jnp.broadcast_to(jnp.where(pos, 0.0, -1.0),
                                     (dinv.shape[0], F))
    hs_ref[...] = dinvF * x_ref[...]


def _tc_prep(degp, x, bn=1024):
    N, F = x.shape
    grid = (N // bn,)
    nf = pl.BlockSpec((bn, F), lambda i: (i, 0))
    return pl.pallas_call(
        _prep_body,
        grid=grid,
        in_specs=[
            pl.BlockSpec((NC, bn, 8), lambda i: (0, i, 0)),
            nf,
        ],
        out_specs=[nf, nf, nf],
        out_shape=[
            jax.ShapeDtypeStruct((N, F), jnp.float32),
            jax.ShapeDtypeStruct((N, F), jnp.float32),
            jax.ShapeDtypeStruct((N, F), jnp.float32),
        ],
    )(degp, x)


def _comb_first_body(accp_ref, h_ref, dinv_ref, diag_ref, tx_ref, hs_ref):
    full = accp_ref[0] + accp_ref[1]
    p = diag_ref[...] * h_ref[...] - dinv_ref[...] * full
    tx_ref[...] = p
    hs_ref[...] = dinv_ref[...] * p


def _comb_rec_body(accp_ref, h_ref, tm2_ref, dinv_ref, diag_ref, tx_ref,
                   hs_ref):
    full = accp_ref[0] + accp_ref[1]
    p = (2.0 * (diag_ref[...] * h_ref[...] - dinv_ref[...] * full)
         - tm2_ref[...])
    tx_ref[...] = p
    hs_ref[...] = dinv_ref[...] * p


def _tc_combine(accp, h, tm2, dinvF, diagF, bn=176, W=2048):
    N, F = h.shape
    R = N * F // W
    wide = lambda a: a.reshape(R, W)
    grid = (R // bn,)
    nf = pl.BlockSpec((bn, W), lambda i: (i, 0))
    accs = pl.BlockSpec((NC, bn, W), lambda i: (0, i, 0))
    if tm2 is None:
        body = _comb_first_body
        ins = (accp.reshape(NC, R, W), wide(h), wide(dinvF), wide(diagF))
        specs = [accs, nf, nf, nf]
    else:
        body = _comb_rec_body
        ins = (accp.reshape(NC, R, W), wide(h), wide(tm2), wide(dinvF),
               wide(diagF))
        specs = [accs, nf, nf, nf, nf]
    tx, hs = pl.pallas_call(
        body,
        grid=grid,
        in_specs=specs,
        out_specs=[nf, nf],
        out_shape=[
            jax.ShapeDtypeStruct((R, W), jnp.float32),
            jax.ShapeDtypeStruct((R, W), jnp.float32),
        ],
    )(*ins)
    return tx.reshape(N, F), hs.reshape(N, F)


def _make_comb_mm_body(K, relu):
    def body(*refs):
        (accp_ref, h_ref, tm2_ref, dinv_ref, diag_ref) = refs[:5]
        t_refs = refs[5:5 + K - 2]
        w_ref, b_ref, o_ref = refs[5 + K - 2:]
        full = accp_ref[0] + accp_ref[1]
        txK = (2.0 * (diag_ref[...] * h_ref[...] - dinv_ref[...] * full)
               - tm2_ref[...])
        acc = b_ref[...]
        for k in range(K - 2):
            acc = acc + jnp.dot(t_refs[k][...], w_ref[k],
                                preferred_element_type=jnp.float32)
        acc = acc + jnp.dot(tm2_ref[...], w_ref[K - 2],
                            preferred_element_type=jnp.float32)
        acc = acc + jnp.dot(h_ref[...], w_ref[K - 1],
                            preferred_element_type=jnp.float32)
        acc = acc + jnp.dot(txK, w_ref[K],
                            preferred_element_type=jnp.float32)
        o_ref[...] = jnp.maximum(acc, 0.0) if relu else acc

    return body


def _tc_comb_mm(accp, txs, dinvF, diagF, W, b, relu, bn=1024):
    N, Fin = txs[0].shape
    K, _, Fout = W.shape
    grid = (N // bn,)
    nf = pl.BlockSpec((bn, Fin), lambda i: (i, 0))
    ins = ([accp, txs[-1], txs[-2], dinvF, diagF]
           + list(txs[:-2]) + [W, b.reshape(1, Fout)])
    specs = ([pl.BlockSpec((NC, bn, Fin), lambda i: (0, i, 0)), nf, nf,
              nf, nf] + [nf] * len(txs[:-2])
             + [pl.BlockSpec((K, Fin, Fout), lambda i: (0, 0, 0)),
                pl.BlockSpec((1, Fout), lambda i: (0, 0))])
    return pl.pallas_call(
        _make_comb_mm_body(K - 1, relu),
        grid=grid,
        in_specs=specs,
        out_specs=pl.BlockSpec((bn, Fout), lambda i: (i, 0)),
        out_shape=jax.ShapeDtypeStruct((N, Fout), jnp.float32),
    )(*ins)


def _level(x, up_idx, edge_index, W, b, relu):
    Nin, Fin = x.shape
    N = up_idx.shape[0]
    E = edge_index.shape[1]
    K = W.shape[0]
    src = edge_index[0].reshape(E // 128, 128)
    dst = edge_index[1].reshape(E // 128, 128)

    zerosF = jnp.zeros((N, Fin), jnp.float32)
    ones8 = jnp.ones((128, 8), jnp.float32)
    zeros8 = jnp.zeros((N, 8), jnp.float32)
    x, degp = _make_sc_gather_deg(Nin, N, Fin, E)(x, up_idx, src, ones8,
                                                  zeros8)

    dinvF, diagF, hs = _tc_prep(degp, x)

    prop = _make_sc_prop(N, E, Fin)
    txs = [x]
    accp = prop(hs, src, dst, zerosF)
    tx, hs = _tc_combine(accp, x, None, dinvF, diagF)
    txs.append(tx)
    for _ in range(2, K - 1):
        accp = prop(hs, src, dst, zerosF)
        tx, hs = _tc_combine(accp, txs[-1], txs[-2], dinvF, diagF)
        txs.append(tx)

    accp = prop(hs, src, dst, zerosF)
    return _tc_comb_mm(accp, txs, dinvF, diagF, W, b, relu)


def kernel(data, up_idx_4to3, edge_index_3, up_idx_3to2, edge_index_2,
           up_idx_2to1, edge_index_1, up_idx_1to0, edge_index_0,
           fc_W, fc_b, W1, b1, W2, b2, W3, b3, W4, b4):
    x = _tc_fc(data, fc_W, fc_b)
    x = x.reshape(-1, 32)
    x = _level(x, up_idx_4to3, edge_index_3, W1, b1, True)
    x = _level(x, up_idx_3to2, edge_index_2, W2, b2, True)
    x = _level(x, up_idx_2to1, edge_index_1, W3, b3, True)
    x = _level(x, up_idx_1to0, edge_index_0, W4, b4, False)
    return x
